# Initial kernel scaffold; baseline (speedup 1.0000x reference)
import jax
import jax.numpy as jnp
from jax.experimental import pallas as pl


def kernel(x_user, x_app, edge_ff, rev_src, rev_dst, Wu, bu, Wa, ba, ff1_Wl, ff1_bl, ff1_Wr, rev1_Wl, rev1_bl, rev1_Wr, rby1_Wl, rby1_bl, rby1_Wr, ff2_Wl, ff2_bl, ff2_Wr, rev2_Wl, rev2_bl, rev2_Wr, rby2_Wl, rby2_bl, rby2_Wr):
    raise NotImplementedError("write your pallas kernel here")


# speedup vs baseline: 1.2558x; 1.2558x over previous
import functools

import jax
import jax.numpy as jnp
from jax import lax
from jax.experimental import pallas as pl
from jax.experimental.pallas import tpu as pltpu
from jax.experimental.pallas import tpu_sc as plsc

_NU, _NA = 50000, 10000
_H, _O = 128, 64
_NU_P, _NA_P = 50048, 10016
_NC, _NS = 2, 16
_NT = _NC * _NS
_K = 128
_W = 16
_RPT_U = _NU_P // _NS
_RPT_A = _NA_P // _NS
_BM = 2000


def _relu(x):
  return jnp.maximum(x, 0.0)


def _full_spec(shape):
  nd = len(shape)
  return pl.BlockSpec(shape, lambda i: (0,) * nd)


def _row_spec(bm, n):
  return pl.BlockSpec((bm, n), lambda i: (i, 0))


def _part_spec(bm, n):
  return pl.BlockSpec((2, bm, n), lambda i: (0, i, 0))


def _chunks(x, n):
  return [x[:, _W * c:_W * (c + 1)] for c in range(n // _W)]


def _k1_user(x_user, Wu, bu, ff1_Wl, rev1_Wl, Wr_u1):
  def body(x_ref, wu_ref, bu_ref, wff_ref, wrev_ref, wru_ref, *outs):
    h = _relu(jnp.dot(x_ref[...], wu_ref[...],
                      preferred_element_type=jnp.float32) + bu_ref[...])
    mff = jnp.dot(h, wff_ref[...], preferred_element_type=jnp.float32)
    mrev = jnp.dot(h, wrev_ref[...], preferred_element_type=jnp.float32)
    for o, val in zip(outs, _chunks(mff, _H) + _chunks(mrev, _H)
                      + [jnp.dot(h, wru_ref[...],
                                 preferred_element_type=jnp.float32)]):
      o[...] = val

  nb = _NU // _BM
  nch = _H // _W
  return pl.pallas_call(
      body,
      grid=(nb,),
      in_specs=[
          _row_spec(_BM, _H), _full_spec((_H, _H)), _full_spec((1, _H)),
          _full_spec((_H, _H)), _full_spec((_H, _H)), _full_spec((_H, _H)),
      ],
      out_specs=[_row_spec(_BM, _W)] * (2 * nch) + [_row_spec(_BM, _H)],
      out_shape=[jax.ShapeDtypeStruct((_NU, _W), jnp.float32)] * (2 * nch)
      + [jax.ShapeDtypeStruct((_NU, _H), jnp.float32)],
  )(x_user, Wu, bu, ff1_Wl, rev1_Wl, Wr_u1)


def _k1_app(x_app, Wa, ba, rby1_Wl, rev1_Wr):
  def body(x_ref, wa_ref, ba_ref, wrby_ref, wsa_ref, *outs):
    h = _relu(jnp.dot(x_ref[...], wa_ref[...],
                      preferred_element_type=jnp.float32) + ba_ref[...])
    m = jnp.dot(h, wrby_ref[...], preferred_element_type=jnp.float32)
    for o, val in zip(outs, _chunks(m, _H)
                      + [jnp.dot(h, wsa_ref[...],
                                 preferred_element_type=jnp.float32)]):
      o[...] = val

  nb = _NA // _BM
  nch = _H // _W
  return pl.pallas_call(
      body,
      grid=(nb,),
      in_specs=[
          _row_spec(_BM, 256), _full_spec((256, _H)), _full_spec((1, _H)),
          _full_spec((_H, _H)), _full_spec((_H, _H)),
      ],
      out_specs=[_row_spec(_BM, _W)] * nch + [_row_spec(_BM, _H)],
      out_shape=[jax.ShapeDtypeStruct((_NA, _W), jnp.float32)] * nch
      + [jax.ShapeDtypeStruct((_NA, _H), jnp.float32)],
  )(x_app, Wa, ba, rby1_Wl, rev1_Wr)


def _inv_from_parts(c_ref):
  cnt = c_ref[0, :, 0:1] + c_ref[1, :, 0:1]
  return 1.0 / jnp.maximum(cnt, 1.0)


def _k2_user(s_ff, s_rby, cnt_ff, cnt_rby, self_u, b_u1, ff2_Wl, rev2_Wl, Wr_u2):
  def body(sff_ref, srby_ref, cff_ref, crby_ref, self_ref, b_ref,
           wff_ref, wrev_ref, wru_ref, *outs):
    sff = sff_ref[0] + sff_ref[1]
    srby = srby_ref[0] + srby_ref[1]
    u = _relu(sff * _inv_from_parts(cff_ref) + srby * _inv_from_parts(crby_ref)
              + b_ref[...] + self_ref[...])
    mff = jnp.dot(u, wff_ref[...], preferred_element_type=jnp.float32)
    mrev = jnp.dot(u, wrev_ref[...], preferred_element_type=jnp.float32)
    for o, val in zip(outs, _chunks(mff, _O) + _chunks(mrev, _O)
                      + [jnp.dot(u, wru_ref[...],
                                 preferred_element_type=jnp.float32)]):
      o[...] = val

  nb = _NU // _BM
  nch = _O // _W
  return pl.pallas_call(
      body,
      grid=(nb,),
      in_specs=[
          _part_spec(_BM, _H), _part_spec(_BM, _H),
          _part_spec(_BM, 16), _part_spec(_BM, 16),
          _row_spec(_BM, _H), _full_spec((1, _H)),
          _full_spec((_H, _O)), _full_spec((_H, _O)), _full_spec((_H, _O)),
      ],
      out_specs=[_row_spec(_BM, _W)] * (2 * nch) + [_row_spec(_BM, _O)],
      out_shape=[jax.ShapeDtypeStruct((_NU, _W), jnp.float32)] * (2 * nch)
      + [jax.ShapeDtypeStruct((_NU, _O), jnp.float32)],
  )(s_ff, s_rby, cnt_ff, cnt_rby, self_u, b_u1, ff2_Wl, rev2_Wl, Wr_u2)


def _k2_app(s_rev, cnt_rev, self_a, b_a1, rby2_Wl, rev2_Wr):
  def body(s_ref, c_ref, self_ref, b_ref, wrby_ref, wsa_ref, *outs):
    s = s_ref[0] + s_ref[1]
    a = _relu(s * _inv_from_parts(c_ref) + b_ref[...] + self_ref[...])
    m = jnp.dot(a, wrby_ref[...], preferred_element_type=jnp.float32)
    for o, val in zip(outs, _chunks(m, _O)
                      + [jnp.dot(a, wsa_ref[...],
                                 preferred_element_type=jnp.float32)]):
      o[...] = val

  nb = _NA // _BM
  nch = _O // _W
  return pl.pallas_call(
      body,
      grid=(nb,),
      in_specs=[
          _part_spec(_BM, _H), _part_spec(_BM, 16),
          _row_spec(_BM, _H), _full_spec((1, _H)),
          _full_spec((_H, _O)), _full_spec((_H, _O)),
      ],
      out_specs=[_row_spec(_BM, _W)] * nch + [_row_spec(_BM, _O)],
      out_shape=[jax.ShapeDtypeStruct((_NA, _W), jnp.float32)] * nch
      + [jax.ShapeDtypeStruct((_NA, _O), jnp.float32)],
  )(s_rev, cnt_rev, self_a, b_a1, rby2_Wl, rev2_Wr)


def _k3_user(s_ff2, s_rby2, cnt_ff, cnt_rby, self_u2, b_u2):
  def body(sff_ref, srby_ref, cff_ref, crby_ref, self_ref, b_ref, out_ref):
    sff = sff_ref[0] + sff_ref[1]
    srby = srby_ref[0] + srby_ref[1]
    out_ref[...] = _relu(
        sff * _inv_from_parts(cff_ref) + srby * _inv_from_parts(crby_ref)
        + b_ref[...] + self_ref[...])

  nb = _NU // _BM
  return pl.pallas_call(
      body,
      grid=(nb,),
      in_specs=[
          _part_spec(_BM, _O), _part_spec(_BM, _O),
          _part_spec(_BM, 16), _part_spec(_BM, 16),
          _row_spec(_BM, _O), _full_spec((1, _O)),
      ],
      out_specs=_row_spec(_BM, _O),
      out_shape=jax.ShapeDtypeStruct((_NU, _O), jnp.float32),
  )(s_ff2, s_rby2, cnt_ff, cnt_rby, self_u2, b_u2)


def _k3_app(s_rev2, cnt_rev, self_a2, b_a2):
  def body(s_ref, c_ref, self_ref, b_ref, out_ref):
    s = s_ref[0] + s_ref[1]
    out_ref[...] = _relu(
        s * _inv_from_parts(c_ref) + b_ref[...] + self_ref[...])

  nb = _NA // _BM
  return pl.pallas_call(
      body,
      grid=(nb,),
      in_specs=[
          _part_spec(_BM, _O), _part_spec(_BM, 16),
          _row_spec(_BM, _O), _full_spec((1, _O)),
      ],
      out_specs=_row_spec(_BM, _O),
      out_shape=jax.ShapeDtypeStruct((_NA, _O), jnp.float32),
  )(s_rev2, cnt_rev, self_a2, b_a2)


_MESH = plsc.VectorSubcoreMesh(core_axis_name="c", subcore_axis_name="s")
_SC_PARAMS = pltpu.CompilerParams(use_tc_tiling_on_sc=False)


def _tile_ids():
  cid = lax.axis_index("c")
  sid = lax.axis_index("s")
  return cid, sid, cid * _NS + sid


def _scatter_loop(src_h, dst_h, tbl_h, sidx, didx, rows, acc, sem, wid):
  e_per = src_h.shape[0] // _NT
  iters = e_per // _K

  def body(j, carry):
    off = wid * e_per + j * _K
    pltpu.sync_copy(src_h.at[pl.ds(off, _K)], sidx)
    pltpu.sync_copy(dst_h.at[pl.ds(off, _K)], didx)
    pltpu.async_copy(tbl_h.at[sidx], rows, sem).wait()
    pltpu.sync_copy(rows, acc.at[didx], add=True)
    return carry

  lax.fori_loop(0, iters, body, 0)


def _count_loop(dst_h, didx, ones_v, acc, wid):
  e_per = dst_h.shape[0] // _NT
  iters = e_per // _K

  def body(j, carry):
    off = wid * e_per + j * _K
    pltpu.sync_copy(dst_h.at[pl.ds(off, _K)], didx)
    pltpu.sync_copy(ones_v, acc.at[didx], add=True)
    return carry

  lax.fori_loop(0, iters, body, 0)


def _sc_counts(ff_dst, rby_dst, rev_dst, zeros_h, ones_h):
  @functools.partial(
      pl.kernel,
      out_type=[
          jax.ShapeDtypeStruct((_NC, _NU_P, 16), jnp.float32),
          jax.ShapeDtypeStruct((_NC, _NU_P, 16), jnp.float32),
          jax.ShapeDtypeStruct((_NC, _NA_P, 16), jnp.float32),
      ],
      mesh=_MESH,
      scratch_types=[
          pltpu.VMEM((_K,), jnp.int32),
          pltpu.VMEM((_K, 16), jnp.float32),
          pltpu.VMEM((_RPT_U, 16), jnp.float32),
          pltpu.VMEM_SHARED((_NU_P, 16), jnp.float32),
      ],
      compiler_params=_SC_PARAMS,
      name="sc_counts",
  )
  def k(ffd_h, rbyd_h, revd_h, z_h, o_h, cf_h, cr_h, ca_h,
        didx, ones_v, zero_v, acc):
    cid, sid, wid = _tile_ids()
    pltpu.sync_copy(z_h, zero_v)
    pltpu.sync_copy(o_h, ones_v)
    for dst_h, out_h, rpt in ((ffd_h, cf_h, _RPT_U), (rbyd_h, cr_h, _RPT_U),
                              (revd_h, ca_h, _RPT_A)):
      pltpu.sync_copy(zero_v.at[pl.ds(0, rpt)], acc.at[pl.ds(sid * rpt, rpt)])
      plsc.subcore_barrier()
      _count_loop(dst_h, didx, ones_v, acc, wid)
      plsc.subcore_barrier()
      pltpu.sync_copy(acc.at[pl.ds(sid * rpt, rpt)],
                      out_h.at[cid, pl.ds(sid * rpt, rpt)])
      plsc.subcore_barrier()

  return k(ff_dst, rby_dst, rev_dst, zeros_h, ones_h)


def _sc_layer(ff_tbls, rby_tbls, rev_tbls, ff_src, ff_dst, rby_src, rby_dst,
              rv_src, rv_dst, zeros_h, token, wu, wa):
  nf, nr, nv = len(ff_tbls), len(rby_tbls), len(rev_tbls)

  @functools.partial(
      pl.kernel,
      out_type=[
          jax.ShapeDtypeStruct((_NC, _NU_P, wu), jnp.float32),
          jax.ShapeDtypeStruct((_NC, _NU_P, wu), jnp.float32),
          jax.ShapeDtypeStruct((_NC, _NA_P, wa), jnp.float32),
      ],
      mesh=_MESH,
      scratch_types=[
          pltpu.VMEM((_K,), jnp.int32),
          pltpu.VMEM((_K,), jnp.int32),
          pltpu.VMEM((_K, _W), jnp.float32),
          pltpu.VMEM((_RPT_U, _W), jnp.float32),
          pltpu.VMEM_SHARED((_NU_P, _W), jnp.float32),
          pltpu.SemaphoreType.DMA,
      ],
      compiler_params=_SC_PARAMS,
      name=f"sc_layer_w{wu}",
  )
  def k(*refs):
    tff = refs[0:nf]
    trby = refs[nf:nf + nr]
    trev = refs[nf + nr:nf + nr + nv]
    (ffs_h, ffd_h, rbys_h, rbyd_h, rvs_h, rvd_h, z_h, _tok,
     sff_h, srby_h, srev_h) = refs[nf + nr + nv:nf + nr + nv + 11]
    sidx, didx, rows, zero_v, acc, sem = refs[nf + nr + nv + 11:]
    cid, sid, wid = _tile_ids()
    pltpu.sync_copy(z_h, zero_v)

    def subpass(tbl, src_h, dst_h, out_h, col, rpt):
      pltpu.sync_copy(zero_v.at[pl.ds(0, rpt)], acc.at[pl.ds(sid * rpt, rpt)])
      plsc.subcore_barrier()
      _scatter_loop(src_h, dst_h, tbl, sidx, didx, rows, acc, sem, wid)
      plsc.subcore_barrier()
      pltpu.sync_copy(acc.at[pl.ds(sid * rpt, rpt)],
                      out_h.at[cid, pl.ds(sid * rpt, rpt), pl.ds(col, _W)])
      plsc.subcore_barrier()

    for c in range(nf):
      subpass(tff[c], ffs_h, ffd_h, sff_h, _W * c, _RPT_U)
    for c in range(nr):
      subpass(trby[c], rbys_h, rbyd_h, srby_h, _W * c, _RPT_U)
    for c in range(nv):
      subpass(trev[c], rvs_h, rvd_h, srev_h, _W * c, _RPT_A)

  return k(*ff_tbls, *rby_tbls, *rev_tbls, ff_src, ff_dst, rby_src, rby_dst,
           rv_src, rv_dst, zeros_h, token)


def _pad_edges(src, dst, junk):
  e = src.shape[0]
  ep = -(-e // (_NT * _K)) * (_NT * _K)
  src_p = jnp.concatenate(
      [src.astype(jnp.int32), jnp.zeros((ep - e,), jnp.int32)])
  dst_p = jnp.concatenate(
      [dst.astype(jnp.int32), jnp.full((ep - e,), junk, jnp.int32)])
  return src_p, dst_p


def kernel(x_user, x_app, edge_ff, rev_src, rev_dst, Wu, bu, Wa, ba,
           ff1_Wl, ff1_bl, ff1_Wr, rev1_Wl, rev1_bl, rev1_Wr,
           rby1_Wl, rby1_bl, rby1_Wr, ff2_Wl, ff2_bl, ff2_Wr,
           rev2_Wl, rev2_bl, rev2_Wr, rby2_Wl, rby2_bl, rby2_Wr):
  ff_src, ff_dst = _pad_edges(edge_ff[0], edge_ff[1], _NU)
  rby_src, rby_dst = _pad_edges(rev_dst, rev_src, _NU)
  rv_src, rv_dst = _pad_edges(rev_src, rev_dst, _NA)

  zeros_u = jnp.zeros((_RPT_U, _W), jnp.float32)
  ones_h = jnp.ones((_K, 16), jnp.float32)

  Wr_u1 = ff1_Wr + rby1_Wr
  Wr_u2 = ff2_Wr + rby2_Wr
  b_u1 = (ff1_bl + rby1_bl).reshape(1, _H)
  b_u2 = (ff2_bl + rby2_bl).reshape(1, _O)
  b_a1 = rev1_bl.reshape(1, _H)
  b_a2 = rev2_bl.reshape(1, _O)

  cnt_ff, cnt_rby, cnt_rev = _sc_counts(ff_dst, rby_dst, rv_dst,
                                        zeros_u, ones_h)

  nch = _H // _W
  k1u = _k1_user(x_user, Wu, bu.reshape(1, _H), ff1_Wl, rev1_Wl, Wr_u1)
  mff_tbls, mrev_tbls, self_u = k1u[:nch], k1u[nch:2 * nch], k1u[-1]
  k1a = _k1_app(x_app, Wa, ba.reshape(1, _H), rby1_Wl, rev1_Wr)
  mrby_tbls, self_a = k1a[:nch], k1a[-1]

  s_ff, s_rby, s_rev = _sc_layer(
      mff_tbls, mrby_tbls, mrev_tbls, ff_src, ff_dst, rby_src, rby_dst,
      rv_src, rv_dst, zeros_u, cnt_rev, _H, _H)

  nch2 = _O // _W
  k2u = _k2_user(s_ff, s_rby, cnt_ff, cnt_rby, self_u, b_u1,
                 ff2_Wl, rev2_Wl, Wr_u2)
  f2_tbls, r2v_tbls, self_u2 = k2u[:nch2], k2u[nch2:2 * nch2], k2u[-1]
  k2a = _k2_app(s_rev, cnt_rev, self_a, b_a1, rby2_Wl, rev2_Wr)
  r2_tbls, self_a2 = k2a[:nch2], k2a[-1]

  s_ff2, s_rby2, s_rev2 = _sc_layer(
      f2_tbls, r2_tbls, r2v_tbls, ff_src, ff_dst, rby_src, rby_dst,
      rv_src, rv_dst, zeros_u, cnt_rev, _O, _O)

  u2 = _k3_user(s_ff2, s_rby2, cnt_ff, cnt_rby, self_u2, b_u2)
  a2 = _k3_app(s_rev2, cnt_rev, self_a2, b_a2)
  return u2, a2


# speedup vs baseline: 1.9814x; 1.5779x over previous
import functools

import jax
import jax.numpy as jnp
from jax import lax
from jax.experimental import pallas as pl
from jax.experimental.pallas import tpu as pltpu
from jax.experimental.pallas import tpu_sc as plsc

_NU, _NA = 50000, 10000
_H, _O = 128, 64
_NU_P, _NA_P = 50048, 10016
_NC, _NS = 2, 16
_NT = _NC * _NS
_K = 128
_W = 16
_RPT_U = _NU_P // _NS
_RPT_A = _NA_P // _NS
_BM = 2000


def _relu(x):
  return jnp.maximum(x, 0.0)


def _full_spec(shape):
  nd = len(shape)
  return pl.BlockSpec(shape, lambda i: (0,) * nd)


def _row_spec(bm, n):
  return pl.BlockSpec((bm, n), lambda i: (i, 0))


def _part_spec(bm, n):
  return pl.BlockSpec((2, bm, n), lambda i: (0, i, 0))


def _chunks(x, n):
  return [x[:, _W * c:_W * (c + 1)] for c in range(n // _W)]


def _k1_user(x_user, Wu, bu, ff1_Wl, rev1_Wl, Wr_u1):
  def body(x_ref, wu_ref, bu_ref, wff_ref, wrev_ref, wru_ref, *outs):
    h = _relu(jnp.dot(x_ref[...], wu_ref[...],
                      preferred_element_type=jnp.float32) + bu_ref[...])
    mff = jnp.dot(h, wff_ref[...], preferred_element_type=jnp.float32)
    mrev = jnp.dot(h, wrev_ref[...], preferred_element_type=jnp.float32)
    for o, val in zip(outs, _chunks(mff, _H) + _chunks(mrev, _H)
                      + [jnp.dot(h, wru_ref[...],
                                 preferred_element_type=jnp.float32)]):
      o[...] = val

  nb = _NU // _BM
  nch = _H // _W
  return pl.pallas_call(
      body,
      grid=(nb,),
      in_specs=[
          _row_spec(_BM, _H), _full_spec((_H, _H)), _full_spec((1, _H)),
          _full_spec((_H, _H)), _full_spec((_H, _H)), _full_spec((_H, _H)),
      ],
      out_specs=[_row_spec(_BM, _W)] * (2 * nch) + [_row_spec(_BM, _H)],
      out_shape=[jax.ShapeDtypeStruct((_NU, _W), jnp.float32)] * (2 * nch)
      + [jax.ShapeDtypeStruct((_NU, _H), jnp.float32)],
  )(x_user, Wu, bu, ff1_Wl, rev1_Wl, Wr_u1)


def _k1_app(x_app, Wa, ba, rby1_Wl, rev1_Wr):
  def body(x_ref, wa_ref, ba_ref, wrby_ref, wsa_ref, *outs):
    h = _relu(jnp.dot(x_ref[...], wa_ref[...],
                      preferred_element_type=jnp.float32) + ba_ref[...])
    m = jnp.dot(h, wrby_ref[...], preferred_element_type=jnp.float32)
    for o, val in zip(outs, _chunks(m, _H)
                      + [jnp.dot(h, wsa_ref[...],
                                 preferred_element_type=jnp.float32)]):
      o[...] = val

  nb = _NA // _BM
  nch = _H // _W
  return pl.pallas_call(
      body,
      grid=(nb,),
      in_specs=[
          _row_spec(_BM, 256), _full_spec((256, _H)), _full_spec((1, _H)),
          _full_spec((_H, _H)), _full_spec((_H, _H)),
      ],
      out_specs=[_row_spec(_BM, _W)] * nch + [_row_spec(_BM, _H)],
      out_shape=[jax.ShapeDtypeStruct((_NA, _W), jnp.float32)] * nch
      + [jax.ShapeDtypeStruct((_NA, _H), jnp.float32)],
  )(x_app, Wa, ba, rby1_Wl, rev1_Wr)


def _inv_from_parts(c_ref):
  cnt = c_ref[0, :, 0:1] + c_ref[1, :, 0:1]
  return 1.0 / jnp.maximum(cnt, 1.0)


def _k2_user(s_ff, s_rby, cnt_ff, cnt_rby, self_u, b_u1, ff2_Wl, rev2_Wl, Wr_u2):
  def body(sff_ref, srby_ref, cff_ref, crby_ref, self_ref, b_ref,
           wff_ref, wrev_ref, wru_ref, *outs):
    sff = sff_ref[0] + sff_ref[1]
    srby = srby_ref[0] + srby_ref[1]
    u = _relu(sff * _inv_from_parts(cff_ref) + srby * _inv_from_parts(crby_ref)
              + b_ref[...] + self_ref[...])
    mff = jnp.dot(u, wff_ref[...], preferred_element_type=jnp.float32)
    mrev = jnp.dot(u, wrev_ref[...], preferred_element_type=jnp.float32)
    for o, val in zip(outs, _chunks(mff, _O) + _chunks(mrev, _O)
                      + [jnp.dot(u, wru_ref[...],
                                 preferred_element_type=jnp.float32)]):
      o[...] = val

  nb = _NU // _BM
  nch = _O // _W
  return pl.pallas_call(
      body,
      grid=(nb,),
      in_specs=[
          _part_spec(_BM, _H), _part_spec(_BM, _H),
          _part_spec(_BM, 16), _part_spec(_BM, 16),
          _row_spec(_BM, _H), _full_spec((1, _H)),
          _full_spec((_H, _O)), _full_spec((_H, _O)), _full_spec((_H, _O)),
      ],
      out_specs=[_row_spec(_BM, _W)] * (2 * nch) + [_row_spec(_BM, _O)],
      out_shape=[jax.ShapeDtypeStruct((_NU, _W), jnp.float32)] * (2 * nch)
      + [jax.ShapeDtypeStruct((_NU, _O), jnp.float32)],
  )(s_ff, s_rby, cnt_ff, cnt_rby, self_u, b_u1, ff2_Wl, rev2_Wl, Wr_u2)


def _k2_app(s_rev, cnt_rev, self_a, b_a1, rby2_Wl, rev2_Wr):
  def body(s_ref, c_ref, self_ref, b_ref, wrby_ref, wsa_ref, *outs):
    s = s_ref[0] + s_ref[1]
    a = _relu(s * _inv_from_parts(c_ref) + b_ref[...] + self_ref[...])
    m = jnp.dot(a, wrby_ref[...], preferred_element_type=jnp.float32)
    for o, val in zip(outs, _chunks(m, _O)
                      + [jnp.dot(a, wsa_ref[...],
                                 preferred_element_type=jnp.float32)]):
      o[...] = val

  nb = _NA // _BM
  nch = _O // _W
  return pl.pallas_call(
      body,
      grid=(nb,),
      in_specs=[
          _part_spec(_BM, _H), _part_spec(_BM, 16),
          _row_spec(_BM, _H), _full_spec((1, _H)),
          _full_spec((_H, _O)), _full_spec((_H, _O)),
      ],
      out_specs=[_row_spec(_BM, _W)] * nch + [_row_spec(_BM, _O)],
      out_shape=[jax.ShapeDtypeStruct((_NA, _W), jnp.float32)] * nch
      + [jax.ShapeDtypeStruct((_NA, _O), jnp.float32)],
  )(s_rev, cnt_rev, self_a, b_a1, rby2_Wl, rev2_Wr)


def _k3_user(s_ff2, s_rby2, cnt_ff, cnt_rby, self_u2, b_u2):
  def body(sff_ref, srby_ref, cff_ref, crby_ref, self_ref, b_ref, out_ref):
    sff = sff_ref[0] + sff_ref[1]
    srby = srby_ref[0] + srby_ref[1]
    out_ref[...] = _relu(
        sff * _inv_from_parts(cff_ref) + srby * _inv_from_parts(crby_ref)
        + b_ref[...] + self_ref[...])

  nb = _NU // _BM
  return pl.pallas_call(
      body,
      grid=(nb,),
      in_specs=[
          _part_spec(_BM, _O), _part_spec(_BM, _O),
          _part_spec(_BM, 16), _part_spec(_BM, 16),
          _row_spec(_BM, _O), _full_spec((1, _O)),
      ],
      out_specs=_row_spec(_BM, _O),
      out_shape=jax.ShapeDtypeStruct((_NU, _O), jnp.float32),
  )(s_ff2, s_rby2, cnt_ff, cnt_rby, self_u2, b_u2)


def _k3_app(s_rev2, cnt_rev, self_a2, b_a2):
  def body(s_ref, c_ref, self_ref, b_ref, out_ref):
    s = s_ref[0] + s_ref[1]
    out_ref[...] = _relu(
        s * _inv_from_parts(c_ref) + b_ref[...] + self_ref[...])

  nb = _NA // _BM
  return pl.pallas_call(
      body,
      grid=(nb,),
      in_specs=[
          _part_spec(_BM, _O), _part_spec(_BM, 16),
          _row_spec(_BM, _O), _full_spec((1, _O)),
      ],
      out_specs=_row_spec(_BM, _O),
      out_shape=jax.ShapeDtypeStruct((_NA, _O), jnp.float32),
  )(s_rev2, cnt_rev, self_a2, b_a2)


_MESH = plsc.VectorSubcoreMesh(core_axis_name="c", subcore_axis_name="s")
_SC_PARAMS = pltpu.CompilerParams(use_tc_tiling_on_sc=False)


def _tile_ids():
  cid = lax.axis_index("c")
  sid = lax.axis_index("s")
  return cid, sid, cid * _NS + sid


def _edge_pipe(tbl_h, sidx2, didx2, rows0, rows1, acc, gsem, ssem, ipt):
  pltpu.async_copy(tbl_h.at[sidx2.at[0]], rows0, gsem)
  npairs = ipt // 2

  def pair(jj, carry):
    j = jj * 2

    @pl.when(jj > 0)
    def _wait_odd_scatter():
      pltpu.make_async_copy(rows1, acc.at[didx2.at[0]], ssem).wait()

    pltpu.async_copy(tbl_h.at[sidx2.at[j + 1]], rows1, gsem)
    pltpu.make_async_copy(tbl_h.at[sidx2.at[j]], rows0, gsem).wait()
    pltpu.async_copy(rows0, acc.at[didx2.at[j]], ssem, add=True)

    pltpu.make_async_copy(rows0, acc.at[didx2.at[0]], ssem).wait()

    @pl.when(jj + 1 < npairs)
    def _next_gather():
      pltpu.async_copy(tbl_h.at[sidx2.at[j + 2]], rows0, gsem)

    pltpu.make_async_copy(tbl_h.at[sidx2.at[j + 1]], rows1, gsem).wait()
    pltpu.async_copy(rows1, acc.at[didx2.at[j + 1]], ssem, add=True)
    return carry

  lax.fori_loop(0, npairs, pair, 0)
  pltpu.make_async_copy(rows1, acc.at[didx2.at[0]], ssem).wait()


def _count_pipe(didx2, ones_v, acc, ssem, ipt):
  def body(j, carry):
    @pl.when(j >= 2)
    def _wait():
      pltpu.make_async_copy(ones_v, acc.at[didx2.at[0]], ssem).wait()

    pltpu.async_copy(ones_v, acc.at[didx2.at[j]], ssem, add=True)
    return carry

  lax.fori_loop(0, ipt, body, 0)
  pltpu.make_async_copy(ones_v, acc.at[didx2.at[0]], ssem).wait()
  pltpu.make_async_copy(ones_v, acc.at[didx2.at[0]], ssem).wait()


def _sc_counts(ff_dst, rby_dst, rev_dst, zeros_h, ones_h):
  ipt_ff = ff_dst.shape[0] // _NT
  ipt_rv = rev_dst.shape[0] // _NT

  @functools.partial(
      pl.kernel,
      out_type=[
          jax.ShapeDtypeStruct((_NC, _NU_P, 16), jnp.float32),
          jax.ShapeDtypeStruct((_NC, _NU_P, 16), jnp.float32),
          jax.ShapeDtypeStruct((_NC, _NA_P, 16), jnp.float32),
      ],
      mesh=_MESH,
      scratch_types=[
          pltpu.VMEM((max(ipt_ff, ipt_rv), _K), jnp.int32),
          pltpu.VMEM((_K, 16), jnp.float32),
          pltpu.VMEM((_RPT_U, 16), jnp.float32),
          pltpu.VMEM_SHARED((_NU_P, 16), jnp.float32),
          pltpu.SemaphoreType.DMA,
      ],
      compiler_params=_SC_PARAMS,
      name="sc_counts",
  )
  def k(ffd_h, rbyd_h, revd_h, z_h, o_h, cf_h, cr_h, ca_h,
        didx2, ones_v, zero_v, acc, ssem):
    cid, sid, wid = _tile_ids()
    pltpu.sync_copy(z_h, zero_v)
    pltpu.sync_copy(o_h, ones_v)
    for dst_h, out_h, rpt, ipt in (
        (ffd_h, cf_h, _RPT_U, ipt_ff), (rbyd_h, cr_h, _RPT_U, ipt_rv),
        (revd_h, ca_h, _RPT_A, ipt_rv)):
      pltpu.sync_copy(dst_h.at[pl.ds(wid * ipt, ipt)], didx2.at[pl.ds(0, ipt)])
      pltpu.sync_copy(zero_v.at[pl.ds(0, rpt)], acc.at[pl.ds(sid * rpt, rpt)])
      plsc.subcore_barrier()
      _count_pipe(didx2, ones_v, acc, ssem, ipt)
      plsc.subcore_barrier()
      pltpu.sync_copy(acc.at[pl.ds(sid * rpt, rpt)],
                      out_h.at[cid, pl.ds(sid * rpt, rpt)])
      plsc.subcore_barrier()

  return k(ff_dst, rby_dst, rev_dst, zeros_h, ones_h)


def _sc_layer(ff_tbls, rby_tbls, rev_tbls, ff_src, ff_dst, rby_src, rby_dst,
              rv_src, rv_dst, zeros_h, token, wu, wa):
  nf, nr, nv = len(ff_tbls), len(rby_tbls), len(rev_tbls)
  ipt_ff = ff_src.shape[0] // _NT
  ipt_rv = rv_src.shape[0] // _NT

  @functools.partial(
      pl.kernel,
      out_type=[
          jax.ShapeDtypeStruct((_NC, _NU_P, wu), jnp.float32),
          jax.ShapeDtypeStruct((_NC, _NU_P, wu), jnp.float32),
          jax.ShapeDtypeStruct((_NC, _NA_P, wa), jnp.float32),
      ],
      mesh=_MESH,
      scratch_types=[
          pltpu.VMEM((max(ipt_ff, ipt_rv), _K), jnp.int32),
          pltpu.VMEM((max(ipt_ff, ipt_rv), _K), jnp.int32),
          pltpu.VMEM((_K, _W), jnp.float32),
          pltpu.VMEM((_K, _W), jnp.float32),
          pltpu.VMEM((_RPT_U, _W), jnp.float32),
          pltpu.VMEM_SHARED((_NU_P, _W), jnp.float32),
          pltpu.SemaphoreType.DMA,
          pltpu.SemaphoreType.DMA,
      ],
      compiler_params=_SC_PARAMS,
      name=f"sc_layer_w{wu}",
  )
  def k(*refs):
    tff = refs[0:nf]
    trby = refs[nf:nf + nr]
    trev = refs[nf + nr:nf + nr + nv]
    (ffs_h, ffd_h, rbys_h, rbyd_h, rvs_h, rvd_h, z_h, _tok,
     sff_h, srby_h, srev_h) = refs[nf + nr + nv:nf + nr + nv + 11]
    sidx2, didx2, rows0, rows1, zero_v, acc, gsem, ssem = \
        refs[nf + nr + nv + 11:]
    cid, sid, wid = _tile_ids()
    pltpu.sync_copy(z_h, zero_v)

    def subpass(tbl, out_h, col, rpt, ipt):
      pltpu.sync_copy(zero_v.at[pl.ds(0, rpt)], acc.at[pl.ds(sid * rpt, rpt)])
      plsc.subcore_barrier()
      _edge_pipe(tbl, sidx2, didx2, rows0, rows1, acc, gsem, ssem, ipt)
      plsc.subcore_barrier()
      pltpu.sync_copy(acc.at[pl.ds(sid * rpt, rpt)],
                      out_h.at[cid, pl.ds(sid * rpt, rpt), pl.ds(col, _W)])
      plsc.subcore_barrier()

    for tbls, src_h, dst_h, out_h, rpt, ipt in (
        (tff, ffs_h, ffd_h, sff_h, _RPT_U, ipt_ff),
        (trby, rbys_h, rbyd_h, srby_h, _RPT_U, ipt_rv),
        (trev, rvs_h, rvd_h, srev_h, _RPT_A, ipt_rv)):
      pltpu.sync_copy(src_h.at[pl.ds(wid * ipt, ipt)], sidx2.at[pl.ds(0, ipt)])
      pltpu.sync_copy(dst_h.at[pl.ds(wid * ipt, ipt)], didx2.at[pl.ds(0, ipt)])
      for c, tbl in enumerate(tbls):
        subpass(tbl, out_h, _W * c, rpt, ipt)

  return k(*ff_tbls, *rby_tbls, *rev_tbls, ff_src, ff_dst, rby_src, rby_dst,
           rv_src, rv_dst, zeros_h, token)


def _pad_edges(src, dst, junk):
  e = src.shape[0]
  ep = -(-e // (2 * _NT * _K)) * (2 * _NT * _K)
  src_p = jnp.concatenate(
      [src.astype(jnp.int32), jnp.zeros((ep - e,), jnp.int32)])
  dst_p = jnp.concatenate(
      [dst.astype(jnp.int32), jnp.full((ep - e,), junk, jnp.int32)])
  return src_p.reshape(-1, _K), dst_p.reshape(-1, _K)


def kernel(x_user, x_app, edge_ff, rev_src, rev_dst, Wu, bu, Wa, ba,
           ff1_Wl, ff1_bl, ff1_Wr, rev1_Wl, rev1_bl, rev1_Wr,
           rby1_Wl, rby1_bl, rby1_Wr, ff2_Wl, ff2_bl, ff2_Wr,
           rev2_Wl, rev2_bl, rev2_Wr, rby2_Wl, rby2_bl, rby2_Wr):
  ff_src, ff_dst = _pad_edges(edge_ff[0], edge_ff[1], _NU)
  rby_src, rby_dst = _pad_edges(rev_dst, rev_src, _NU)
  rv_src, rv_dst = _pad_edges(rev_src, rev_dst, _NA)

  zeros_u = jnp.zeros((_RPT_U, _W), jnp.float32)
  ones_h = jnp.ones((_K, 16), jnp.float32)

  Wr_u1 = ff1_Wr + rby1_Wr
  Wr_u2 = ff2_Wr + rby2_Wr
  b_u1 = (ff1_bl + rby1_bl).reshape(1, _H)
  b_u2 = (ff2_bl + rby2_bl).reshape(1, _O)
  b_a1 = rev1_bl.reshape(1, _H)
  b_a2 = rev2_bl.reshape(1, _O)

  cnt_ff, cnt_rby, cnt_rev = _sc_counts(ff_dst, rby_dst, rv_dst,
                                        zeros_u, ones_h)

  nch = _H // _W
  k1u = _k1_user(x_user, Wu, bu.reshape(1, _H), ff1_Wl, rev1_Wl, Wr_u1)
  mff_tbls, mrev_tbls, self_u = k1u[:nch], k1u[nch:2 * nch], k1u[-1]
  k1a = _k1_app(x_app, Wa, ba.reshape(1, _H), rby1_Wl, rev1_Wr)
  mrby_tbls, self_a = k1a[:nch], k1a[-1]

  s_ff, s_rby, s_rev = _sc_layer(
      mff_tbls, mrby_tbls, mrev_tbls, ff_src, ff_dst, rby_src, rby_dst,
      rv_src, rv_dst, zeros_u, cnt_rev, _H, _H)

  nch2 = _O // _W
  k2u = _k2_user(s_ff, s_rby, cnt_ff, cnt_rby, self_u, b_u1,
                 ff2_Wl, rev2_Wl, Wr_u2)
  f2_tbls, r2v_tbls, self_u2 = k2u[:nch2], k2u[nch2:2 * nch2], k2u[-1]
  k2a = _k2_app(s_rev, cnt_rev, self_a, b_a1, rby2_Wl, rev2_Wr)
  r2_tbls, self_a2 = k2a[:nch2], k2a[-1]

  s_ff2, s_rby2, s_rev2 = _sc_layer(
      f2_tbls, r2_tbls, r2v_tbls, ff_src, ff_dst, rby_src, rby_dst,
      rv_src, rv_dst, zeros_u, cnt_rev, _O, _O)

  u2 = _k3_user(s_ff2, s_rby2, cnt_ff, cnt_rby, self_u2, b_u2)
  a2 = _k3_app(s_rev2, cnt_rev, self_a2, b_a2)
  return u2, a2


# speedup vs baseline: 2.4187x; 1.2207x over previous
import functools

import jax
import jax.numpy as jnp
from jax import lax
from jax.experimental import pallas as pl
from jax.experimental.pallas import tpu as pltpu
from jax.experimental.pallas import tpu_sc as plsc

_NU, _NA = 50000, 10000
_H, _O = 128, 64
_NU_P, _NA_P = 50048, 10016
_NC, _NS = 2, 16
_NT = _NC * _NS
_K = 128
_W = 16
_RPT_U = _NU_P // _NS
_RPT_A = _NA_P // _NS
_BM = 2000


def _relu(x):
  return jnp.maximum(x, 0.0)


def _full_spec(shape):
  nd = len(shape)
  return pl.BlockSpec(shape, lambda i: (0,) * nd)


def _row_spec(bm, n):
  return pl.BlockSpec((bm, n), lambda i: (i, 0))


def _part_spec(bm, n):
  return pl.BlockSpec((2, bm, n), lambda i: (0, i, 0))


def _chunks(x, n):
  return [x[:, _W * c:_W * (c + 1)] for c in range(n // _W)]


def _k1_user(x_user, Wu, bu, ff1_Wl, rev1_Wl, Wr_u1):
  def body(x_ref, wu_ref, bu_ref, wff_ref, wrev_ref, wru_ref, *outs):
    h = _relu(jnp.dot(x_ref[...], wu_ref[...],
                      preferred_element_type=jnp.float32) + bu_ref[...])
    mff = jnp.dot(h, wff_ref[...], preferred_element_type=jnp.float32)
    mrev = jnp.dot(h, wrev_ref[...], preferred_element_type=jnp.float32)
    for o, val in zip(outs, _chunks(mff, _H) + _chunks(mrev, _H)
                      + [jnp.dot(h, wru_ref[...],
                                 preferred_element_type=jnp.float32)]):
      o[...] = val

  nb = _NU // _BM
  nch = _H // _W
  return pl.pallas_call(
      body,
      grid=(nb,),
      in_specs=[
          _row_spec(_BM, _H), _full_spec((_H, _H)), _full_spec((1, _H)),
          _full_spec((_H, _H)), _full_spec((_H, _H)), _full_spec((_H, _H)),
      ],
      out_specs=[_row_spec(_BM, _W)] * (2 * nch) + [_row_spec(_BM, _H)],
      out_shape=[jax.ShapeDtypeStruct((_NU, _W), jnp.float32)] * (2 * nch)
      + [jax.ShapeDtypeStruct((_NU, _H), jnp.float32)],
  )(x_user, Wu, bu, ff1_Wl, rev1_Wl, Wr_u1)


def _k1_app(x_app, Wa, ba, rby1_Wl, rev1_Wr):
  def body(x_ref, wa_ref, ba_ref, wrby_ref, wsa_ref, *outs):
    h = _relu(jnp.dot(x_ref[...], wa_ref[...],
                      preferred_element_type=jnp.float32) + ba_ref[...])
    m = jnp.dot(h, wrby_ref[...], preferred_element_type=jnp.float32)
    for o, val in zip(outs, _chunks(m, _H)
                      + [jnp.dot(h, wsa_ref[...],
                                 preferred_element_type=jnp.float32)]):
      o[...] = val

  nb = _NA // _BM
  nch = _H // _W
  return pl.pallas_call(
      body,
      grid=(nb,),
      in_specs=[
          _row_spec(_BM, 256), _full_spec((256, _H)), _full_spec((1, _H)),
          _full_spec((_H, _H)), _full_spec((_H, _H)),
      ],
      out_specs=[_row_spec(_BM, _W)] * nch + [_row_spec(_BM, _H)],
      out_shape=[jax.ShapeDtypeStruct((_NA, _W), jnp.float32)] * nch
      + [jax.ShapeDtypeStruct((_NA, _H), jnp.float32)],
  )(x_app, Wa, ba, rby1_Wl, rev1_Wr)


def _inv_from_parts(c_ref):
  cnt = c_ref[0, :, 0:1] + c_ref[1, :, 0:1]
  return 1.0 / jnp.maximum(cnt, 1.0)


def _k2_user(s_ff, s_rby, cnt_ff, cnt_rby, self_u, b_u1, ff2_Wl, rev2_Wl, Wr_u2):
  def body(sff_ref, srby_ref, cff_ref, crby_ref, self_ref, b_ref,
           wff_ref, wrev_ref, wru_ref, *outs):
    sff = sff_ref[0] + sff_ref[1]
    srby = srby_ref[0] + srby_ref[1]
    u = _relu(sff * _inv_from_parts(cff_ref) + srby * _inv_from_parts(crby_ref)
              + b_ref[...] + self_ref[...])
    mff = jnp.dot(u, wff_ref[...], preferred_element_type=jnp.float32)
    mrev = jnp.dot(u, wrev_ref[...], preferred_element_type=jnp.float32)
    for o, val in zip(outs, _chunks(mff, _O) + _chunks(mrev, _O)
                      + [jnp.dot(u, wru_ref[...],
                                 preferred_element_type=jnp.float32)]):
      o[...] = val

  nb = _NU // _BM
  nch = _O // _W
  return pl.pallas_call(
      body,
      grid=(nb,),
      in_specs=[
          _part_spec(_BM, _H), _part_spec(_BM, _H),
          _part_spec(_BM, 16), _part_spec(_BM, 16),
          _row_spec(_BM, _H), _full_spec((1, _H)),
          _full_spec((_H, _O)), _full_spec((_H, _O)), _full_spec((_H, _O)),
      ],
      out_specs=[_row_spec(_BM, _W)] * (2 * nch) + [_row_spec(_BM, _O)],
      out_shape=[jax.ShapeDtypeStruct((_NU, _W), jnp.float32)] * (2 * nch)
      + [jax.ShapeDtypeStruct((_NU, _O), jnp.float32)],
  )(s_ff, s_rby, cnt_ff, cnt_rby, self_u, b_u1, ff2_Wl, rev2_Wl, Wr_u2)


def _k2_app(s_rev, cnt_rev, self_a, b_a1, rby2_Wl, rev2_Wr):
  def body(s_ref, c_ref, self_ref, b_ref, wrby_ref, wsa_ref, *outs):
    s = s_ref[0] + s_ref[1]
    a = _relu(s * _inv_from_parts(c_ref) + b_ref[...] + self_ref[...])
    m = jnp.dot(a, wrby_ref[...], preferred_element_type=jnp.float32)
    for o, val in zip(outs, _chunks(m, _O)
                      + [jnp.dot(a, wsa_ref[...],
                                 preferred_element_type=jnp.float32)]):
      o[...] = val

  nb = _NA // _BM
  nch = _O // _W
  return pl.pallas_call(
      body,
      grid=(nb,),
      in_specs=[
          _part_spec(_BM, _H), _part_spec(_BM, 16),
          _row_spec(_BM, _H), _full_spec((1, _H)),
          _full_spec((_H, _O)), _full_spec((_H, _O)),
      ],
      out_specs=[_row_spec(_BM, _W)] * nch + [_row_spec(_BM, _O)],
      out_shape=[jax.ShapeDtypeStruct((_NA, _W), jnp.float32)] * nch
      + [jax.ShapeDtypeStruct((_NA, _O), jnp.float32)],
  )(s_rev, cnt_rev, self_a, b_a1, rby2_Wl, rev2_Wr)


def _k3_user(s_ff2, s_rby2, cnt_ff, cnt_rby, self_u2, b_u2):
  def body(sff_ref, srby_ref, cff_ref, crby_ref, self_ref, b_ref, out_ref):
    sff = sff_ref[0] + sff_ref[1]
    srby = srby_ref[0] + srby_ref[1]
    out_ref[...] = _relu(
        sff * _inv_from_parts(cff_ref) + srby * _inv_from_parts(crby_ref)
        + b_ref[...] + self_ref[...])

  nb = _NU // _BM
  return pl.pallas_call(
      body,
      grid=(nb,),
      in_specs=[
          _part_spec(_BM, _O), _part_spec(_BM, _O),
          _part_spec(_BM, 16), _part_spec(_BM, 16),
          _row_spec(_BM, _O), _full_spec((1, _O)),
      ],
      out_specs=_row_spec(_BM, _O),
      out_shape=jax.ShapeDtypeStruct((_NU, _O), jnp.float32),
  )(s_ff2, s_rby2, cnt_ff, cnt_rby, self_u2, b_u2)


def _k3_app(s_rev2, cnt_rev, self_a2, b_a2):
  def body(s_ref, c_ref, self_ref, b_ref, out_ref):
    s = s_ref[0] + s_ref[1]
    out_ref[...] = _relu(
        s * _inv_from_parts(c_ref) + b_ref[...] + self_ref[...])

  nb = _NA // _BM
  return pl.pallas_call(
      body,
      grid=(nb,),
      in_specs=[
          _part_spec(_BM, _O), _part_spec(_BM, 16),
          _row_spec(_BM, _O), _full_spec((1, _O)),
      ],
      out_specs=_row_spec(_BM, _O),
      out_shape=jax.ShapeDtypeStruct((_NA, _O), jnp.float32),
  )(s_rev2, cnt_rev, self_a2, b_a2)


_MESH = plsc.VectorSubcoreMesh(core_axis_name="c", subcore_axis_name="s")
_SC_PARAMS = pltpu.CompilerParams(use_tc_tiling_on_sc=False)


def _tile_ids():
  cid = lax.axis_index("c")
  sid = lax.axis_index("s")
  return cid, sid, cid * _NS + sid


def _edge_pipe(tbl_h, sidx2, didx2, rows0, rows1, acc, gsem, ssem, ipt):
  pltpu.async_copy(tbl_h.at[sidx2.at[0]], rows0, gsem)
  npairs = ipt // 2

  def pair(jj, carry):
    j = jj * 2

    @pl.when(jj > 0)
    def _wait_odd_scatter():
      pltpu.make_async_copy(rows1, acc.at[didx2.at[0]], ssem).wait()

    pltpu.async_copy(tbl_h.at[sidx2.at[j + 1]], rows1, gsem)
    pltpu.make_async_copy(tbl_h.at[sidx2.at[j]], rows0, gsem).wait()
    pltpu.async_copy(rows0, acc.at[didx2.at[j]], ssem, add=True)

    pltpu.make_async_copy(rows0, acc.at[didx2.at[0]], ssem).wait()

    @pl.when(jj + 1 < npairs)
    def _next_gather():
      pltpu.async_copy(tbl_h.at[sidx2.at[j + 2]], rows0, gsem)

    pltpu.make_async_copy(tbl_h.at[sidx2.at[j + 1]], rows1, gsem).wait()
    pltpu.async_copy(rows1, acc.at[didx2.at[j + 1]], ssem, add=True)
    return carry

  lax.fori_loop(0, npairs, pair, 0)
  pltpu.make_async_copy(rows1, acc.at[didx2.at[0]], ssem).wait()


def _count_pipe(didx2, ones_v, acc, ssem, ipt):
  def body(j, carry):
    @pl.when(j >= 2)
    def _wait():
      pltpu.make_async_copy(ones_v, acc.at[didx2.at[0]], ssem).wait()

    pltpu.async_copy(ones_v, acc.at[didx2.at[j]], ssem, add=True)
    return carry

  lax.fori_loop(0, ipt, body, 0)
  pltpu.make_async_copy(ones_v, acc.at[didx2.at[0]], ssem).wait()
  pltpu.make_async_copy(ones_v, acc.at[didx2.at[0]], ssem).wait()


def _sc_counts(ff_dst, rby_dst, rev_dst, zeros_h, ones_h):
  ipt_ff = ff_dst.shape[0] // _NT
  ipt_rv = rev_dst.shape[0] // _NT

  @functools.partial(
      pl.kernel,
      out_type=[
          jax.ShapeDtypeStruct((_NC, _NU_P, 16), jnp.float32),
          jax.ShapeDtypeStruct((_NC, _NU_P, 16), jnp.float32),
          jax.ShapeDtypeStruct((_NC, _NA_P, 16), jnp.float32),
      ],
      mesh=_MESH,
      scratch_types=[
          pltpu.VMEM((max(ipt_ff, ipt_rv), _K), jnp.int32),
          pltpu.VMEM((_K, 16), jnp.float32),
          pltpu.VMEM((_RPT_U, 16), jnp.float32),
          pltpu.VMEM_SHARED((_NU_P, 16), jnp.float32),
          pltpu.SemaphoreType.DMA,
      ],
      compiler_params=_SC_PARAMS,
      name="sc_counts",
  )
  def k(ffd_h, rbyd_h, revd_h, z_h, o_h, cf_h, cr_h, ca_h,
        didx2, ones_v, zero_v, acc, ssem):
    cid, sid, wid = _tile_ids()
    pltpu.sync_copy(z_h, zero_v)
    pltpu.sync_copy(o_h, ones_v)
    for dst_h, out_h, rpt, ipt in (
        (ffd_h, cf_h, _RPT_U, ipt_ff), (rbyd_h, cr_h, _RPT_U, ipt_rv),
        (revd_h, ca_h, _RPT_A, ipt_rv)):
      pltpu.sync_copy(dst_h.at[pl.ds(wid * ipt, ipt)], didx2.at[pl.ds(0, ipt)])
      pltpu.sync_copy(zero_v.at[pl.ds(0, rpt)], acc.at[pl.ds(sid * rpt, rpt)])
      plsc.subcore_barrier()
      _count_pipe(didx2, ones_v, acc, ssem, ipt)
      plsc.subcore_barrier()
      pltpu.sync_copy(acc.at[pl.ds(sid * rpt, rpt)],
                      out_h.at[cid, pl.ds(sid * rpt, rpt)])
      plsc.subcore_barrier()

  return k(ff_dst, rby_dst, rev_dst, zeros_h, ones_h)


def _sc_layer(ff_tbls, rby_tbls, rev_tbls, ff_src, ff_dst, rby_src, rby_dst,
              rv_src, rv_dst, zeros_h, token, wu, wa):
  nf, nr, nv = len(ff_tbls), len(rby_tbls), len(rev_tbls)
  ipt_ff = ff_src.shape[0] // _NT
  ipt_rv = rv_src.shape[0] // _NT

  @functools.partial(
      pl.kernel,
      out_type=[
          jax.ShapeDtypeStruct((_NC, _NU_P, wu), jnp.float32),
          jax.ShapeDtypeStruct((_NC, _NU_P, wu), jnp.float32),
          jax.ShapeDtypeStruct((_NC, _NA_P, wa), jnp.float32),
      ],
      mesh=_MESH,
      scratch_types=[
          pltpu.VMEM((max(ipt_ff, ipt_rv), _K), jnp.int32),
          pltpu.VMEM((max(ipt_ff, ipt_rv), _K), jnp.int32),
          pltpu.VMEM((_K, _W), jnp.float32),
          pltpu.VMEM((_K, _W), jnp.float32),
          pltpu.VMEM((_RPT_U, _W), jnp.float32),
          pltpu.VMEM_SHARED((_NU_P, _W), jnp.float32),
          pltpu.SemaphoreType.DMA,
          pltpu.SemaphoreType.DMA,
      ],
      compiler_params=_SC_PARAMS,
      name=f"sc_layer_w{wu}",
  )
  def k(*refs):
    tff = refs[0:nf]
    trby = refs[nf:nf + nr]
    trev = refs[nf + nr:nf + nr + nv]
    (ffs_h, ffd_h, rbys_h, rbyd_h, rvs_h, rvd_h, z_h, _tok,
     sff_h, srby_h, srev_h) = refs[nf + nr + nv:nf + nr + nv + 11]
    sidx2, didx2, rows0, rows1, zero_v, acc, gsem, ssem = \
        refs[nf + nr + nv + 11:]
    cid, sid, wid = _tile_ids()
    pltpu.sync_copy(z_h, zero_v)

    def subpass(tbl, out_h, col, rpt, ipt):
      pltpu.sync_copy(zero_v.at[pl.ds(0, rpt)], acc.at[pl.ds(sid * rpt, rpt)])
      plsc.subcore_barrier()
      _edge_pipe(tbl, sidx2, didx2, rows0, rows1, acc, gsem, ssem, ipt)
      plsc.subcore_barrier()
      pltpu.sync_copy(acc.at[pl.ds(sid * rpt, rpt)],
                      out_h.at[cid, pl.ds(sid * rpt, rpt), pl.ds(col, _W)])
      plsc.subcore_barrier()

    for tbls, src_h, dst_h, out_h, rpt, ipt in (
        (tff, ffs_h, ffd_h, sff_h, _RPT_U, ipt_ff),
        (trby, rbys_h, rbyd_h, srby_h, _RPT_U, ipt_rv),
        (trev, rvs_h, rvd_h, srev_h, _RPT_A, ipt_rv)):
      pltpu.sync_copy(src_h.at[pl.ds(wid * ipt, ipt)], sidx2.at[pl.ds(0, ipt)])
      pltpu.sync_copy(dst_h.at[pl.ds(wid * ipt, ipt)], didx2.at[pl.ds(0, ipt)])
      for c, tbl in enumerate(tbls):
        subpass(tbl, out_h, _W * c, rpt, ipt)

  return k(*ff_tbls, *rby_tbls, *rev_tbls, ff_src, ff_dst, rby_src, rby_dst,
           rv_src, rv_dst, zeros_h, token)


def _pad_edges(src, dst, junk, njunk):
  e = src.shape[0]
  ep = -(-e // (2 * _NT * _K)) * (2 * _NT * _K)
  pad = jnp.arange(ep - e, dtype=jnp.int32)
  src_p = jnp.concatenate([src.astype(jnp.int32), pad % 997])
  dst_p = jnp.concatenate([dst.astype(jnp.int32), junk + pad % njunk])
  return src_p.reshape(-1, _K), dst_p.reshape(-1, _K)


def kernel(x_user, x_app, edge_ff, rev_src, rev_dst, Wu, bu, Wa, ba,
           ff1_Wl, ff1_bl, ff1_Wr, rev1_Wl, rev1_bl, rev1_Wr,
           rby1_Wl, rby1_bl, rby1_Wr, ff2_Wl, ff2_bl, ff2_Wr,
           rev2_Wl, rev2_bl, rev2_Wr, rby2_Wl, rby2_bl, rby2_Wr):
  ff_src, ff_dst = _pad_edges(edge_ff[0], edge_ff[1], _NU, _NU_P - _NU)
  rby_src, rby_dst = _pad_edges(rev_dst, rev_src, _NU, _NU_P - _NU)
  rv_src, rv_dst = _pad_edges(rev_src, rev_dst, _NA, _NA_P - _NA)

  zeros_u = jnp.zeros((_RPT_U, _W), jnp.float32)
  ones_h = jnp.ones((_K, 16), jnp.float32)

  Wr_u1 = ff1_Wr + rby1_Wr
  Wr_u2 = ff2_Wr + rby2_Wr
  b_u1 = (ff1_bl + rby1_bl).reshape(1, _H)
  b_u2 = (ff2_bl + rby2_bl).reshape(1, _O)
  b_a1 = rev1_bl.reshape(1, _H)
  b_a2 = rev2_bl.reshape(1, _O)

  cnt_ff, cnt_rby, cnt_rev = _sc_counts(ff_dst, rby_dst, rv_dst,
                                        zeros_u, ones_h)

  nch = _H // _W
  k1u = _k1_user(x_user, Wu, bu.reshape(1, _H), ff1_Wl, rev1_Wl, Wr_u1)
  mff_tbls, mrev_tbls, self_u = k1u[:nch], k1u[nch:2 * nch], k1u[-1]
  k1a = _k1_app(x_app, Wa, ba.reshape(1, _H), rby1_Wl, rev1_Wr)
  mrby_tbls, self_a = k1a[:nch], k1a[-1]

  s_ff, s_rby, s_rev = _sc_layer(
      mff_tbls, mrby_tbls, mrev_tbls, ff_src, ff_dst, rby_src, rby_dst,
      rv_src, rv_dst, zeros_u, cnt_rev, _H, _H)

  nch2 = _O // _W
  k2u = _k2_user(s_ff, s_rby, cnt_ff, cnt_rby, self_u, b_u1,
                 ff2_Wl, rev2_Wl, Wr_u2)
  f2_tbls, r2v_tbls, self_u2 = k2u[:nch2], k2u[nch2:2 * nch2], k2u[-1]
  k2a = _k2_app(s_rev, cnt_rev, self_a, b_a1, rby2_Wl, rev2_Wr)
  r2_tbls, self_a2 = k2a[:nch2], k2a[-1]

  s_ff2, s_rby2, s_rev2 = _sc_layer(
      f2_tbls, r2_tbls, r2v_tbls, ff_src, ff_dst, rby_src, rby_dst,
      rv_src, rv_dst, zeros_u, cnt_rev, _O, _O)

  u2 = _k3_user(s_ff2, s_rby2, cnt_ff, cnt_rby, self_u2, b_u2)
  a2 = _k3_app(s_rev2, cnt_rev, self_a2, b_a2)
  return u2, a2


# speedup vs baseline: 2.7268x; 1.1274x over previous
import functools

import jax
import jax.numpy as jnp
from jax import lax
from jax.experimental import pallas as pl
from jax.experimental.pallas import tpu as pltpu
from jax.experimental.pallas import tpu_sc as plsc

_NU, _NA = 50000, 10000
_H, _O = 128, 64
_NU_P, _NA_P = 50048, 10016
_NC, _NS = 2, 16
_NT = _NC * _NS
_K = 128
_W = 16
_RPT_U = _NU_P // _NS
_RPT_A = _NA_P // _NS
_BM = 2000


def _relu(x):
  return jnp.maximum(x, 0.0)


def _full_spec(shape):
  nd = len(shape)
  return pl.BlockSpec(shape, lambda i: (0,) * nd)


def _row_spec(bm, n):
  return pl.BlockSpec((bm, n), lambda i: (i, 0))


def _part_spec(bm, n):
  return pl.BlockSpec((2, bm, n), lambda i: (0, i, 0))


def _chunks(x, n):
  return [x[:, _W * c:_W * (c + 1)] for c in range(n // _W)]


def _k1_user(x_user, Wu, bu, ff1_Wl, rev1_Wl, Wr_u1):
  def body(x_ref, wu_ref, bu_ref, wff_ref, wrev_ref, wru_ref, *outs):
    h = _relu(jnp.dot(x_ref[...], wu_ref[...],
                      preferred_element_type=jnp.float32) + bu_ref[...])
    mff = jnp.dot(h, wff_ref[...], preferred_element_type=jnp.float32)
    mrev = jnp.dot(h, wrev_ref[...], preferred_element_type=jnp.float32)
    for o, val in zip(outs, _chunks(mff, _H) + _chunks(mrev, _H)
                      + [jnp.dot(h, wru_ref[...],
                                 preferred_element_type=jnp.float32)]):
      o[...] = val

  nb = _NU // _BM
  nch = _H // _W
  return pl.pallas_call(
      body,
      grid=(nb,),
      in_specs=[
          _row_spec(_BM, _H), _full_spec((_H, _H)), _full_spec((1, _H)),
          _full_spec((_H, _H)), _full_spec((_H, _H)), _full_spec((_H, _H)),
      ],
      out_specs=[_row_spec(_BM, _W)] * (2 * nch) + [_row_spec(_BM, _H)],
      out_shape=[jax.ShapeDtypeStruct((_NU, _W), jnp.float32)] * (2 * nch)
      + [jax.ShapeDtypeStruct((_NU, _H), jnp.float32)],
  )(x_user, Wu, bu, ff1_Wl, rev1_Wl, Wr_u1)


def _k1_app(x_app, Wa, ba, rby1_Wl, rev1_Wr):
  def body(x_ref, wa_ref, ba_ref, wrby_ref, wsa_ref, *outs):
    h = _relu(jnp.dot(x_ref[...], wa_ref[...],
                      preferred_element_type=jnp.float32) + ba_ref[...])
    m = jnp.dot(h, wrby_ref[...], preferred_element_type=jnp.float32)
    for o, val in zip(outs, _chunks(m, _H)
                      + [jnp.dot(h, wsa_ref[...],
                                 preferred_element_type=jnp.float32)]):
      o[...] = val

  nb = _NA // _BM
  nch = _H // _W
  return pl.pallas_call(
      body,
      grid=(nb,),
      in_specs=[
          _row_spec(_BM, 256), _full_spec((256, _H)), _full_spec((1, _H)),
          _full_spec((_H, _H)), _full_spec((_H, _H)),
      ],
      out_specs=[_row_spec(_BM, _W)] * nch + [_row_spec(_BM, _H)],
      out_shape=[jax.ShapeDtypeStruct((_NA, _W), jnp.float32)] * nch
      + [jax.ShapeDtypeStruct((_NA, _H), jnp.float32)],
  )(x_app, Wa, ba, rby1_Wl, rev1_Wr)


def _inv_from_parts(c_ref):
  cnt = c_ref[0, :, 0:1] + c_ref[1, :, 0:1]
  return 1.0 / jnp.maximum(cnt, 1.0)


def _k2_user(s_ff, s_rby, cnt_ff, cnt_rby, self_u, b_u1, ff2_Wl, rev2_Wl, Wr_u2):
  def body(sff_ref, srby_ref, cff_ref, crby_ref, self_ref, b_ref,
           wff_ref, wrev_ref, wru_ref, *outs):
    sff = sff_ref[0] + sff_ref[1]
    srby = srby_ref[0] + srby_ref[1]
    u = _relu(sff * _inv_from_parts(cff_ref) + srby * _inv_from_parts(crby_ref)
              + b_ref[...] + self_ref[...])
    mff = jnp.dot(u, wff_ref[...], preferred_element_type=jnp.float32)
    mrev = jnp.dot(u, wrev_ref[...], preferred_element_type=jnp.float32)
    for o, val in zip(outs, _chunks(mff, _O) + _chunks(mrev, _O)
                      + [jnp.dot(u, wru_ref[...],
                                 preferred_element_type=jnp.float32)]):
      o[...] = val

  nb = _NU // _BM
  nch = _O // _W
  return pl.pallas_call(
      body,
      grid=(nb,),
      in_specs=[
          _part_spec(_BM, _H), _part_spec(_BM, _H),
          _part_spec(_BM, 16), _part_spec(_BM, 16),
          _row_spec(_BM, _H), _full_spec((1, _H)),
          _full_spec((_H, _O)), _full_spec((_H, _O)), _full_spec((_H, _O)),
      ],
      out_specs=[_row_spec(_BM, _W)] * (2 * nch) + [_row_spec(_BM, _O)],
      out_shape=[jax.ShapeDtypeStruct((_NU, _W), jnp.float32)] * (2 * nch)
      + [jax.ShapeDtypeStruct((_NU, _O), jnp.float32)],
  )(s_ff, s_rby, cnt_ff, cnt_rby, self_u, b_u1, ff2_Wl, rev2_Wl, Wr_u2)


def _k2_app(s_rev, cnt_rev, self_a, b_a1, rby2_Wl, rev2_Wr):
  def body(s_ref, c_ref, self_ref, b_ref, wrby_ref, wsa_ref, *outs):
    s = s_ref[0] + s_ref[1]
    a = _relu(s * _inv_from_parts(c_ref) + b_ref[...] + self_ref[...])
    m = jnp.dot(a, wrby_ref[...], preferred_element_type=jnp.float32)
    for o, val in zip(outs, _chunks(m, _O)
                      + [jnp.dot(a, wsa_ref[...],
                                 preferred_element_type=jnp.float32)]):
      o[...] = val

  nb = _NA // _BM
  nch = _O // _W
  return pl.pallas_call(
      body,
      grid=(nb,),
      in_specs=[
          _part_spec(_BM, _H), _part_spec(_BM, 16),
          _row_spec(_BM, _H), _full_spec((1, _H)),
          _full_spec((_H, _O)), _full_spec((_H, _O)),
      ],
      out_specs=[_row_spec(_BM, _W)] * nch + [_row_spec(_BM, _O)],
      out_shape=[jax.ShapeDtypeStruct((_NA, _W), jnp.float32)] * nch
      + [jax.ShapeDtypeStruct((_NA, _O), jnp.float32)],
  )(s_rev, cnt_rev, self_a, b_a1, rby2_Wl, rev2_Wr)


def _k3_user(s_ff2, s_rby2, cnt_ff, cnt_rby, self_u2, b_u2):
  def body(sff_ref, srby_ref, cff_ref, crby_ref, self_ref, b_ref, out_ref):
    sff = sff_ref[0] + sff_ref[1]
    srby = srby_ref[0] + srby_ref[1]
    out_ref[...] = _relu(
        sff * _inv_from_parts(cff_ref) + srby * _inv_from_parts(crby_ref)
        + b_ref[...] + self_ref[...])

  nb = _NU // _BM
  return pl.pallas_call(
      body,
      grid=(nb,),
      in_specs=[
          _part_spec(_BM, _O), _part_spec(_BM, _O),
          _part_spec(_BM, 16), _part_spec(_BM, 16),
          _row_spec(_BM, _O), _full_spec((1, _O)),
      ],
      out_specs=_row_spec(_BM, _O),
      out_shape=jax.ShapeDtypeStruct((_NU, _O), jnp.float32),
  )(s_ff2, s_rby2, cnt_ff, cnt_rby, self_u2, b_u2)


def _k3_app(s_rev2, cnt_rev, self_a2, b_a2):
  def body(s_ref, c_ref, self_ref, b_ref, out_ref):
    s = s_ref[0] + s_ref[1]
    out_ref[...] = _relu(
        s * _inv_from_parts(c_ref) + b_ref[...] + self_ref[...])

  nb = _NA // _BM
  return pl.pallas_call(
      body,
      grid=(nb,),
      in_specs=[
          _part_spec(_BM, _O), _part_spec(_BM, 16),
          _row_spec(_BM, _O), _full_spec((1, _O)),
      ],
      out_specs=_row_spec(_BM, _O),
      out_shape=jax.ShapeDtypeStruct((_NA, _O), jnp.float32),
  )(s_rev2, cnt_rev, self_a2, b_a2)


_MESH = plsc.VectorSubcoreMesh(core_axis_name="c", subcore_axis_name="s")
_SC_PARAMS = pltpu.CompilerParams(use_tc_tiling_on_sc=False)


def _tile_ids():
  cid = lax.axis_index("c")
  sid = lax.axis_index("s")
  return cid, sid, cid * _NS + sid


def _edge_pipe(tbl_h, sidx2, didx2, rows0, rows1, acc, gsem, ssem, ipt):
  pltpu.async_copy(tbl_h.at[sidx2.at[0]], rows0, gsem)
  npairs = ipt // 2

  def pair(jj, carry):
    j = jj * 2

    @pl.when(jj > 0)
    def _wait_odd_scatter():
      pltpu.make_async_copy(rows1, acc.at[didx2.at[0]], ssem).wait()

    pltpu.async_copy(tbl_h.at[sidx2.at[j + 1]], rows1, gsem)
    pltpu.make_async_copy(tbl_h.at[sidx2.at[j]], rows0, gsem).wait()
    pltpu.async_copy(rows0, acc.at[didx2.at[j]], ssem, add=True)

    pltpu.make_async_copy(rows0, acc.at[didx2.at[0]], ssem).wait()

    @pl.when(jj + 1 < npairs)
    def _next_gather():
      pltpu.async_copy(tbl_h.at[sidx2.at[j + 2]], rows0, gsem)

    pltpu.make_async_copy(tbl_h.at[sidx2.at[j + 1]], rows1, gsem).wait()
    pltpu.async_copy(rows1, acc.at[didx2.at[j + 1]], ssem, add=True)
    return carry

  lax.fori_loop(0, npairs, pair, 0)
  pltpu.make_async_copy(rows1, acc.at[didx2.at[0]], ssem).wait()


def _count_pipe(didx2, ones_v, acc, ssem, ipt):
  def body(j, carry):
    @pl.when(j >= 2)
    def _wait():
      pltpu.make_async_copy(ones_v, acc.at[didx2.at[0]], ssem).wait()

    pltpu.async_copy(ones_v, acc.at[didx2.at[j]], ssem, add=True)
    return carry

  lax.fori_loop(0, ipt, body, 0)
  pltpu.make_async_copy(ones_v, acc.at[didx2.at[0]], ssem).wait()
  pltpu.make_async_copy(ones_v, acc.at[didx2.at[0]], ssem).wait()


def _sc_counts(ff_dst, rby_dst, rev_dst, zeros_h, ones_h):
  ipt_ff = ff_dst.shape[0] // _NT
  ipt_rv = rev_dst.shape[0] // _NT

  @functools.partial(
      pl.kernel,
      out_type=[
          jax.ShapeDtypeStruct((_NC, _NU_P, 16), jnp.float32),
          jax.ShapeDtypeStruct((_NC, _NU_P, 16), jnp.float32),
          jax.ShapeDtypeStruct((_NC, _NA_P, 16), jnp.float32),
      ],
      mesh=_MESH,
      scratch_types=[
          pltpu.VMEM((max(ipt_ff, ipt_rv), _K), jnp.int32),
          pltpu.VMEM((_K, 16), jnp.float32),
          pltpu.VMEM((_RPT_U, 16), jnp.float32),
          pltpu.VMEM_SHARED((_NU_P, 16), jnp.float32),
          pltpu.SemaphoreType.DMA,
      ],
      compiler_params=_SC_PARAMS,
      name="sc_counts",
  )
  def k(ffd_h, rbyd_h, revd_h, z_h, o_h, cf_h, cr_h, ca_h,
        didx2, ones_v, zero_v, acc, ssem):
    cid, sid, wid = _tile_ids()
    pltpu.sync_copy(z_h, zero_v)
    pltpu.sync_copy(o_h, ones_v)
    for dst_h, out_h, rpt, ipt in (
        (ffd_h, cf_h, _RPT_U, ipt_ff), (rbyd_h, cr_h, _RPT_U, ipt_rv),
        (revd_h, ca_h, _RPT_A, ipt_rv)):
      pltpu.sync_copy(dst_h.at[pl.ds(wid * ipt, ipt)], didx2.at[pl.ds(0, ipt)])
      pltpu.sync_copy(zero_v.at[pl.ds(0, rpt)], acc.at[pl.ds(sid * rpt, rpt)])
      plsc.subcore_barrier()
      _count_pipe(didx2, ones_v, acc, ssem, ipt)
      plsc.subcore_barrier()
      pltpu.sync_copy(acc.at[pl.ds(sid * rpt, rpt)],
                      out_h.at[cid, pl.ds(sid * rpt, rpt)])
      plsc.subcore_barrier()

  return k(ff_dst, rby_dst, rev_dst, zeros_h, ones_h)


def _sc_agg(groups, zeros_h, token, name):
  counts = [len(tbls) for tbls, _, _, _ in groups]
  ipts = [src2.shape[0] // _NT for _, src2, _, _ in groups]
  ipt_max = max(ipts)
  ntbl = sum(counts)
  ng = len(groups)

  @functools.partial(
      pl.kernel,
      out_type=[
          jax.ShapeDtypeStruct((_NC, rpt * _NS, _W * len(tbls)), jnp.float32)
          for tbls, _, _, rpt in groups
      ],
      mesh=_MESH,
      scratch_types=[
          pltpu.VMEM((ipt_max, _K), jnp.int32),
          pltpu.VMEM((ipt_max, _K), jnp.int32),
          pltpu.VMEM((_K, _W), jnp.float32),
          pltpu.VMEM((_K, _W), jnp.float32),
          pltpu.VMEM((_RPT_U, _W), jnp.float32),
          pltpu.VMEM_SHARED((_NU_P, _W), jnp.float32),
          pltpu.SemaphoreType.DMA,
          pltpu.SemaphoreType.DMA,
      ],
      compiler_params=_SC_PARAMS,
      name=name,
  )
  def k(*refs):
    pos = 0
    tbl_refs = []
    for n in counts:
      tbl_refs.append(refs[pos:pos + n])
      pos += n
    edge_refs = refs[pos:pos + 2 * ng]
    pos += 2 * ng
    z_h = refs[pos]
    out_refs = refs[pos + 2:pos + 2 + ng]
    sidx2, didx2, rows0, rows1, zero_v, acc, gsem, ssem = refs[pos + 2 + ng:]
    cid, sid, wid = _tile_ids()
    pltpu.sync_copy(z_h, zero_v)

    def subpass(tbl, out_h, col, rpt, ipt):
      pltpu.sync_copy(zero_v.at[pl.ds(0, rpt)], acc.at[pl.ds(sid * rpt, rpt)])
      plsc.subcore_barrier()
      _edge_pipe(tbl, sidx2, didx2, rows0, rows1, acc, gsem, ssem, ipt)
      plsc.subcore_barrier()
      pltpu.sync_copy(acc.at[pl.ds(sid * rpt, rpt)],
                      out_h.at[cid, pl.ds(sid * rpt, rpt), pl.ds(col, _W)])
      plsc.subcore_barrier()

    for g in range(ng):
      src_h, dst_h = edge_refs[2 * g], edge_refs[2 * g + 1]
      rpt, ipt = groups[g][3], ipts[g]
      pltpu.sync_copy(src_h.at[pl.ds(wid * ipt, ipt)], sidx2.at[pl.ds(0, ipt)])
      pltpu.sync_copy(dst_h.at[pl.ds(wid * ipt, ipt)], didx2.at[pl.ds(0, ipt)])
      for c, tbl in enumerate(tbl_refs[g]):
        subpass(tbl, out_refs[g], _W * c, rpt, ipt)

  args = []
  for tbls, _, _, _ in groups:
    args.extend(tbls)
  for _, src2, dst2, _ in groups:
    args.extend((src2, dst2))
  args.extend((zeros_h, token))
  out = k(*args)
  return tuple(out) if isinstance(out, (list, tuple)) else (out,)


def _pad_edges(src, dst, junk, njunk):
  e = src.shape[0]
  ep = -(-e // (2 * _NT * _K)) * (2 * _NT * _K)
  pad = jnp.arange(ep - e, dtype=jnp.int32)
  src_p = jnp.concatenate([src.astype(jnp.int32), pad % 997])
  dst_p = jnp.concatenate([dst.astype(jnp.int32), junk + pad % njunk])
  return src_p.reshape(-1, _K), dst_p.reshape(-1, _K)


def kernel(x_user, x_app, edge_ff, rev_src, rev_dst, Wu, bu, Wa, ba,
           ff1_Wl, ff1_bl, ff1_Wr, rev1_Wl, rev1_bl, rev1_Wr,
           rby1_Wl, rby1_bl, rby1_Wr, ff2_Wl, ff2_bl, ff2_Wr,
           rev2_Wl, rev2_bl, rev2_Wr, rby2_Wl, rby2_bl, rby2_Wr):
  ff_src, ff_dst = _pad_edges(edge_ff[0], edge_ff[1], _NU, _NU_P - _NU)
  rby_src, rby_dst = _pad_edges(rev_dst, rev_src, _NU, _NU_P - _NU)
  rv_src, rv_dst = _pad_edges(rev_src, rev_dst, _NA, _NA_P - _NA)

  zeros_u = jnp.zeros((_RPT_U, _W), jnp.float32)
  ones_h = jnp.ones((_K, 16), jnp.float32)

  Wr_u1 = ff1_Wr + rby1_Wr
  Wr_u2 = ff2_Wr + rby2_Wr
  b_u1 = (ff1_bl + rby1_bl).reshape(1, _H)
  b_u2 = (ff2_bl + rby2_bl).reshape(1, _O)
  b_a1 = rev1_bl.reshape(1, _H)
  b_a2 = rev2_bl.reshape(1, _O)

  cnt_ff, cnt_rby, cnt_rev = _sc_counts(ff_dst, rby_dst, rv_dst,
                                        zeros_u, ones_h)

  nch = _H // _W
  k1u = _k1_user(x_user, Wu, bu.reshape(1, _H), ff1_Wl, rev1_Wl, Wr_u1)
  mff_tbls, mrev_tbls, self_u = k1u[:nch], k1u[nch:2 * nch], k1u[-1]
  k1a = _k1_app(x_app, Wa, ba.reshape(1, _H), rby1_Wl, rev1_Wr)
  mrby_tbls, self_a = k1a[:nch], k1a[-1]

  s_ff, s_rby = _sc_agg(
      ((mff_tbls, ff_src, ff_dst, _RPT_U),
       (mrby_tbls, rby_src, rby_dst, _RPT_U)),
      zeros_u, cnt_rev, "sc_l1_users")
  (s_rev,) = _sc_agg(((mrev_tbls, rv_src, rv_dst, _RPT_A),),
                     zeros_u, s_rby, "sc_l1_apps")

  nch2 = _O // _W
  k2u = _k2_user(s_ff, s_rby, cnt_ff, cnt_rby, self_u, b_u1,
                 ff2_Wl, rev2_Wl, Wr_u2)
  f2_tbls, r2v_tbls, self_u2 = k2u[:nch2], k2u[nch2:2 * nch2], k2u[-1]
  (s_ff2,) = _sc_agg(((f2_tbls, ff_src, ff_dst, _RPT_U),),
                     zeros_u, s_rev, "sc_l2_ff")
  k2a = _k2_app(s_rev, cnt_rev, self_a, b_a1, rby2_Wl, rev2_Wr)
  r2_tbls, self_a2 = k2a[:nch2], k2a[-1]

  s_rby2, s_rev2 = _sc_agg(
      ((r2_tbls, rby_src, rby_dst, _RPT_U),
       (r2v_tbls, rv_src, rv_dst, _RPT_A)),
      zeros_u, s_ff2, "sc_l2_rest")

  u2 = _k3_user(s_ff2, s_rby2, cnt_ff, cnt_rby, self_u2, b_u2)
  a2 = _k3_app(s_rev2, cnt_rev, self_a2, b_a2)
  return u2, a2


# speedup vs baseline: 2.9311x; 1.0749x over previous
import functools

import jax
import jax.numpy as jnp
from jax import lax
from jax.experimental import pallas as pl
from jax.experimental.pallas import tpu as pltpu
from jax.experimental.pallas import tpu_sc as plsc

_NU, _NA = 50000, 10000
_H, _O = 128, 64
_NU_P, _NA_P = 50048, 10016
_NC, _NS = 2, 16
_NT = _NC * _NS
_K = 128
_W = 16
_RPT_U = _NU_P // _NS
_RPT_A = _NA_P // _NS
_BM = 2000


def _relu(x):
  return jnp.maximum(x, 0.0)


def _full_spec(shape):
  nd = len(shape)
  return pl.BlockSpec(shape, lambda i: (0,) * nd)


def _row_spec(bm, n):
  return pl.BlockSpec((bm, n), lambda i: (i, 0))


def _part_spec(bm, n):
  return pl.BlockSpec((2, bm, n), lambda i: (0, i, 0))


def _chunks(x, n):
  return [x[:, _W * c:_W * (c + 1)] for c in range(n // _W)]


def _k1_user(x_user, Wu, bu, ff1_Wl, rev1_Wl, Wr_u1):
  def body(x_ref, wu_ref, bu_ref, wff_ref, wrev_ref, wru_ref, *outs):
    h = _relu(jnp.dot(x_ref[...], wu_ref[...],
                      preferred_element_type=jnp.float32) + bu_ref[...])
    mff = jnp.dot(h, wff_ref[...], preferred_element_type=jnp.float32)
    mrev = jnp.dot(h, wrev_ref[...], preferred_element_type=jnp.float32)
    for o, val in zip(outs, _chunks(mff, _H) + _chunks(mrev, _H)
                      + [jnp.dot(h, wru_ref[...],
                                 preferred_element_type=jnp.float32)]):
      o[...] = val

  nb = _NU // _BM
  nch = _H // _W
  return pl.pallas_call(
      body,
      grid=(nb,),
      in_specs=[
          _row_spec(_BM, _H), _full_spec((_H, _H)), _full_spec((1, _H)),
          _full_spec((_H, _H)), _full_spec((_H, _H)), _full_spec((_H, _H)),
      ],
      out_specs=[_row_spec(_BM, _W)] * (2 * nch) + [_row_spec(_BM, _H)],
      out_shape=[jax.ShapeDtypeStruct((_NU, _W), jnp.float32)] * (2 * nch)
      + [jax.ShapeDtypeStruct((_NU, _H), jnp.float32)],
  )(x_user, Wu, bu, ff1_Wl, rev1_Wl, Wr_u1)


def _k1_app(x_app, Wa, ba, rby1_Wl, rev1_Wr):
  def body(x_ref, wa_ref, ba_ref, wrby_ref, wsa_ref, *outs):
    h = _relu(jnp.dot(x_ref[...], wa_ref[...],
                      preferred_element_type=jnp.float32) + ba_ref[...])
    m = jnp.dot(h, wrby_ref[...], preferred_element_type=jnp.float32)
    for o, val in zip(outs, _chunks(m, _H)
                      + [jnp.dot(h, wsa_ref[...],
                                 preferred_element_type=jnp.float32)]):
      o[...] = val

  nb = _NA // _BM
  nch = _H // _W
  return pl.pallas_call(
      body,
      grid=(nb,),
      in_specs=[
          _row_spec(_BM, 256), _full_spec((256, _H)), _full_spec((1, _H)),
          _full_spec((_H, _H)), _full_spec((_H, _H)),
      ],
      out_specs=[_row_spec(_BM, _W)] * nch + [_row_spec(_BM, _H)],
      out_shape=[jax.ShapeDtypeStruct((_NA, _W), jnp.float32)] * nch
      + [jax.ShapeDtypeStruct((_NA, _H), jnp.float32)],
  )(x_app, Wa, ba, rby1_Wl, rev1_Wr)


def _inv_from_parts(c_ref):
  cnt = c_ref[0, :, 0:1] + c_ref[1, :, 0:1]
  return 1.0 / jnp.maximum(cnt, 1.0)


def _k2_user(s_ff, s_rby, cnt_ff, cnt_rby, self_u, b_u1, ff2_Wl, rev2_Wl, Wr_u2):
  def body(sff_ref, srby_ref, cff_ref, crby_ref, self_ref, b_ref,
           wff_ref, wrev_ref, wru_ref, *outs):
    sff = sff_ref[0] + sff_ref[1]
    srby = srby_ref[0] + srby_ref[1]
    u = _relu(sff * _inv_from_parts(cff_ref) + srby * _inv_from_parts(crby_ref)
              + b_ref[...] + self_ref[...])
    mff = jnp.dot(u, wff_ref[...], preferred_element_type=jnp.float32)
    mrev = jnp.dot(u, wrev_ref[...], preferred_element_type=jnp.float32)
    for o, val in zip(outs, _chunks(mff, _O) + _chunks(mrev, _O)
                      + [jnp.dot(u, wru_ref[...],
                                 preferred_element_type=jnp.float32)]):
      o[...] = val

  nb = _NU // _BM
  nch = _O // _W
  return pl.pallas_call(
      body,
      grid=(nb,),
      in_specs=[
          _part_spec(_BM, _H), _part_spec(_BM, _H),
          _part_spec(_BM, 16), _part_spec(_BM, 16),
          _row_spec(_BM, _H), _full_spec((1, _H)),
          _full_spec((_H, _O)), _full_spec((_H, _O)), _full_spec((_H, _O)),
      ],
      out_specs=[_row_spec(_BM, _W)] * (2 * nch) + [_row_spec(_BM, _O)],
      out_shape=[jax.ShapeDtypeStruct((_NU, _W), jnp.float32)] * (2 * nch)
      + [jax.ShapeDtypeStruct((_NU, _O), jnp.float32)],
  )(s_ff, s_rby, cnt_ff, cnt_rby, self_u, b_u1, ff2_Wl, rev2_Wl, Wr_u2)


def _k2_app(s_rev, cnt_rev, self_a, b_a1, rby2_Wl, rev2_Wr):
  def body(s_ref, c_ref, self_ref, b_ref, wrby_ref, wsa_ref, *outs):
    s = s_ref[0] + s_ref[1]
    a = _relu(s * _inv_from_parts(c_ref) + b_ref[...] + self_ref[...])
    m = jnp.dot(a, wrby_ref[...], preferred_element_type=jnp.float32)
    for o, val in zip(outs, _chunks(m, _O)
                      + [jnp.dot(a, wsa_ref[...],
                                 preferred_element_type=jnp.float32)]):
      o[...] = val

  nb = _NA // _BM
  nch = _O // _W
  return pl.pallas_call(
      body,
      grid=(nb,),
      in_specs=[
          _part_spec(_BM, _H), _part_spec(_BM, 16),
          _row_spec(_BM, _H), _full_spec((1, _H)),
          _full_spec((_H, _O)), _full_spec((_H, _O)),
      ],
      out_specs=[_row_spec(_BM, _W)] * nch + [_row_spec(_BM, _O)],
      out_shape=[jax.ShapeDtypeStruct((_NA, _W), jnp.float32)] * nch
      + [jax.ShapeDtypeStruct((_NA, _O), jnp.float32)],
  )(s_rev, cnt_rev, self_a, b_a1, rby2_Wl, rev2_Wr)


def _k3_user(s_ff2, s_rby2, cnt_ff, cnt_rby, self_u2, b_u2):
  def body(sff_ref, srby_ref, cff_ref, crby_ref, self_ref, b_ref, out_ref):
    sff = sff_ref[0] + sff_ref[1]
    srby = srby_ref[0] + srby_ref[1]
    out_ref[...] = _relu(
        sff * _inv_from_parts(cff_ref) + srby * _inv_from_parts(crby_ref)
        + b_ref[...] + self_ref[...])

  nb = _NU // _BM
  return pl.pallas_call(
      body,
      grid=(nb,),
      in_specs=[
          _part_spec(_BM, _O), _part_spec(_BM, _O),
          _part_spec(_BM, 16), _part_spec(_BM, 16),
          _row_spec(_BM, _O), _full_spec((1, _O)),
      ],
      out_specs=_row_spec(_BM, _O),
      out_shape=jax.ShapeDtypeStruct((_NU, _O), jnp.float32),
  )(s_ff2, s_rby2, cnt_ff, cnt_rby, self_u2, b_u2)


def _k3_app(s_rev2, cnt_rev, self_a2, b_a2):
  def body(s_ref, c_ref, self_ref, b_ref, out_ref):
    s = s_ref[0] + s_ref[1]
    out_ref[...] = _relu(
        s * _inv_from_parts(c_ref) + b_ref[...] + self_ref[...])

  nb = _NA // _BM
  return pl.pallas_call(
      body,
      grid=(nb,),
      in_specs=[
          _part_spec(_BM, _O), _part_spec(_BM, 16),
          _row_spec(_BM, _O), _full_spec((1, _O)),
      ],
      out_specs=_row_spec(_BM, _O),
      out_shape=jax.ShapeDtypeStruct((_NA, _O), jnp.float32),
  )(s_rev2, cnt_rev, self_a2, b_a2)


_MESH = plsc.VectorSubcoreMesh(core_axis_name="c", subcore_axis_name="s")
_SC_PARAMS = pltpu.CompilerParams(use_tc_tiling_on_sc=False)


def _tile_ids():
  cid = lax.axis_index("c")
  sid = lax.axis_index("s")
  return cid, sid, cid * _NS + sid


def _edge_pipe(tbl_h, sidx2, didx2, rows, acc, gsems, ssems, ipt):
  pltpu.async_copy(tbl_h.at[sidx2.at[0]], rows[0], gsems[0])
  pltpu.async_copy(tbl_h.at[sidx2.at[1]], rows[1], gsems[1])
  nquads = ipt // 4

  def quad(q, carry):
    for k in range(4):
      j = q * 4 + k
      ge, se = gsems[k % 2], ssems[k % 2]
      buf, nbuf = rows[k], rows[(k + 2) % 4]

      @pl.when(j >= 2)
      def _wait_scatter():
        pltpu.make_async_copy(nbuf, acc.at[didx2.at[0]], se).wait()

      pltpu.make_async_copy(tbl_h.at[sidx2.at[0]], buf, ge).wait()

      @pl.when(j + 2 < ipt)
      def _next_gather():
        pltpu.async_copy(tbl_h.at[sidx2.at[j + 2]], nbuf, ge)

      pltpu.async_copy(buf, acc.at[didx2.at[j]], se, add=True)
    return carry

  lax.fori_loop(0, nquads, quad, 0)
  pltpu.make_async_copy(rows[0], acc.at[didx2.at[0]], ssems[0]).wait()
  pltpu.make_async_copy(rows[1], acc.at[didx2.at[0]], ssems[1]).wait()


def _count_pipe(didx2, ones_v, acc, ssem, ipt):
  def body(j, carry):
    @pl.when(j >= 2)
    def _wait():
      pltpu.make_async_copy(ones_v, acc.at[didx2.at[0]], ssem).wait()

    pltpu.async_copy(ones_v, acc.at[didx2.at[j]], ssem, add=True)
    return carry

  lax.fori_loop(0, ipt, body, 0)
  pltpu.make_async_copy(ones_v, acc.at[didx2.at[0]], ssem).wait()
  pltpu.make_async_copy(ones_v, acc.at[didx2.at[0]], ssem).wait()


def _sc_counts(ff_dst, rby_dst, rev_dst, zeros_h, ones_h):
  ipt_ff = ff_dst.shape[0] // _NT
  ipt_rv = rev_dst.shape[0] // _NT

  @functools.partial(
      pl.kernel,
      out_type=[
          jax.ShapeDtypeStruct((_NC, _NU_P, 16), jnp.float32),
          jax.ShapeDtypeStruct((_NC, _NU_P, 16), jnp.float32),
          jax.ShapeDtypeStruct((_NC, _NA_P, 16), jnp.float32),
      ],
      mesh=_MESH,
      scratch_types=[
          pltpu.VMEM((max(ipt_ff, ipt_rv), _K), jnp.int32),
          pltpu.VMEM((_K, 16), jnp.float32),
          pltpu.VMEM((_RPT_U, 16), jnp.float32),
          pltpu.VMEM_SHARED((_NU_P, 16), jnp.float32),
          pltpu.SemaphoreType.DMA,
      ],
      compiler_params=_SC_PARAMS,
      name="sc_counts",
  )
  def k(ffd_h, rbyd_h, revd_h, z_h, o_h, cf_h, cr_h, ca_h,
        didx2, ones_v, zero_v, acc, ssem):
    cid, sid, wid = _tile_ids()
    pltpu.sync_copy(z_h, zero_v)
    pltpu.sync_copy(o_h, ones_v)
    for dst_h, out_h, rpt, ipt in (
        (ffd_h, cf_h, _RPT_U, ipt_ff), (rbyd_h, cr_h, _RPT_U, ipt_rv),
        (revd_h, ca_h, _RPT_A, ipt_rv)):
      pltpu.sync_copy(dst_h.at[pl.ds(wid * ipt, ipt)], didx2.at[pl.ds(0, ipt)])
      pltpu.sync_copy(zero_v.at[pl.ds(0, rpt)], acc.at[pl.ds(sid * rpt, rpt)])
      plsc.subcore_barrier()
      _count_pipe(didx2, ones_v, acc, ssem, ipt)
      plsc.subcore_barrier()
      pltpu.sync_copy(acc.at[pl.ds(sid * rpt, rpt)],
                      out_h.at[cid, pl.ds(sid * rpt, rpt)])
      plsc.subcore_barrier()

  return k(ff_dst, rby_dst, rev_dst, zeros_h, ones_h)


def _sc_agg(groups, zeros_h, token, name):
  counts = [len(tbls) for tbls, _, _, _ in groups]
  ipts = [src2.shape[0] // _NT for _, src2, _, _ in groups]
  ipt_max = max(ipts)
  ntbl = sum(counts)
  ng = len(groups)

  @functools.partial(
      pl.kernel,
      out_type=[
          jax.ShapeDtypeStruct((_NC, rpt * _NS, _W * len(tbls)), jnp.float32)
          for tbls, _, _, rpt in groups
      ],
      mesh=_MESH,
      scratch_types=[
          pltpu.VMEM((ipt_max, _K), jnp.int32),
          pltpu.VMEM((ipt_max, _K), jnp.int32),
          pltpu.VMEM((_K, _W), jnp.float32),
          pltpu.VMEM((_K, _W), jnp.float32),
          pltpu.VMEM((_K, _W), jnp.float32),
          pltpu.VMEM((_K, _W), jnp.float32),
          pltpu.VMEM((_RPT_U, _W), jnp.float32),
          pltpu.VMEM_SHARED((_NU_P, _W), jnp.float32),
          pltpu.SemaphoreType.DMA,
          pltpu.SemaphoreType.DMA,
          pltpu.SemaphoreType.DMA,
          pltpu.SemaphoreType.DMA,
      ],
      compiler_params=_SC_PARAMS,
      name=name,
  )
  def k(*refs):
    pos = 0
    tbl_refs = []
    for n in counts:
      tbl_refs.append(refs[pos:pos + n])
      pos += n
    edge_refs = refs[pos:pos + 2 * ng]
    pos += 2 * ng
    z_h = refs[pos]
    out_refs = refs[pos + 2:pos + 2 + ng]
    (sidx2, didx2, rows0, rows1, rows2, rows3, zero_v, acc,
     gsem0, gsem1, ssem0, ssem1) = refs[pos + 2 + ng:]
    cid, sid, wid = _tile_ids()
    pltpu.sync_copy(z_h, zero_v)

    def subpass(tbl, out_h, col, rpt, ipt):
      pltpu.sync_copy(zero_v.at[pl.ds(0, rpt)], acc.at[pl.ds(sid * rpt, rpt)])
      plsc.subcore_barrier()
      _edge_pipe(tbl, sidx2, didx2, (rows0, rows1, rows2, rows3), acc,
                 (gsem0, gsem1), (ssem0, ssem1), ipt)
      plsc.subcore_barrier()
      pltpu.sync_copy(acc.at[pl.ds(sid * rpt, rpt)],
                      out_h.at[cid, pl.ds(sid * rpt, rpt), pl.ds(col, _W)])
      plsc.subcore_barrier()

    for g in range(ng):
      src_h, dst_h = edge_refs[2 * g], edge_refs[2 * g + 1]
      rpt, ipt = groups[g][3], ipts[g]
      pltpu.sync_copy(src_h.at[pl.ds(wid * ipt, ipt)], sidx2.at[pl.ds(0, ipt)])
      pltpu.sync_copy(dst_h.at[pl.ds(wid * ipt, ipt)], didx2.at[pl.ds(0, ipt)])
      for c, tbl in enumerate(tbl_refs[g]):
        subpass(tbl, out_refs[g], _W * c, rpt, ipt)

  args = []
  for tbls, _, _, _ in groups:
    args.extend(tbls)
  for _, src2, dst2, _ in groups:
    args.extend((src2, dst2))
  args.extend((zeros_h, token))
  out = k(*args)
  return tuple(out) if isinstance(out, (list, tuple)) else (out,)


def _pad_edges(src, dst, junk, njunk):
  e = src.shape[0]
  ep = -(-e // (4 * _NT * _K)) * (4 * _NT * _K)
  pad = jnp.arange(ep - e, dtype=jnp.int32)
  src_p = jnp.concatenate([src.astype(jnp.int32), pad % 997])
  dst_p = jnp.concatenate([dst.astype(jnp.int32), junk + pad % njunk])
  return src_p.reshape(-1, _K), dst_p.reshape(-1, _K)


def kernel(x_user, x_app, edge_ff, rev_src, rev_dst, Wu, bu, Wa, ba,
           ff1_Wl, ff1_bl, ff1_Wr, rev1_Wl, rev1_bl, rev1_Wr,
           rby1_Wl, rby1_bl, rby1_Wr, ff2_Wl, ff2_bl, ff2_Wr,
           rev2_Wl, rev2_bl, rev2_Wr, rby2_Wl, rby2_bl, rby2_Wr):
  ff_src, ff_dst = _pad_edges(edge_ff[0], edge_ff[1], _NU, _NU_P - _NU)
  rby_src, rby_dst = _pad_edges(rev_dst, rev_src, _NU, _NU_P - _NU)
  rv_src, rv_dst = _pad_edges(rev_src, rev_dst, _NA, _NA_P - _NA)

  zeros_u = jnp.zeros((_RPT_U, _W), jnp.float32)
  ones_h = jnp.ones((_K, 16), jnp.float32)

  Wr_u1 = ff1_Wr + rby1_Wr
  Wr_u2 = ff2_Wr + rby2_Wr
  b_u1 = (ff1_bl + rby1_bl).reshape(1, _H)
  b_u2 = (ff2_bl + rby2_bl).reshape(1, _O)
  b_a1 = rev1_bl.reshape(1, _H)
  b_a2 = rev2_bl.reshape(1, _O)

  cnt_ff, cnt_rby, cnt_rev = _sc_counts(ff_dst, rby_dst, rv_dst,
                                        zeros_u, ones_h)

  nch = _H // _W
  k1u = _k1_user(x_user, Wu, bu.reshape(1, _H), ff1_Wl, rev1_Wl, Wr_u1)
  mff_tbls, mrev_tbls, self_u = k1u[:nch], k1u[nch:2 * nch], k1u[-1]
  k1a = _k1_app(x_app, Wa, ba.reshape(1, _H), rby1_Wl, rev1_Wr)
  mrby_tbls, self_a = k1a[:nch], k1a[-1]

  s_ff, s_rby = _sc_agg(
      ((mff_tbls, ff_src, ff_dst, _RPT_U),
       (mrby_tbls, rby_src, rby_dst, _RPT_U)),
      zeros_u, cnt_rev, "sc_l1_users")
  (s_rev,) = _sc_agg(((mrev_tbls, rv_src, rv_dst, _RPT_A),),
                     zeros_u, s_rby, "sc_l1_apps")

  nch2 = _O // _W
  k2u = _k2_user(s_ff, s_rby, cnt_ff, cnt_rby, self_u, b_u1,
                 ff2_Wl, rev2_Wl, Wr_u2)
  f2_tbls, r2v_tbls, self_u2 = k2u[:nch2], k2u[nch2:2 * nch2], k2u[-1]
  (s_ff2,) = _sc_agg(((f2_tbls, ff_src, ff_dst, _RPT_U),),
                     zeros_u, s_rev, "sc_l2_ff")
  k2a = _k2_app(s_rev, cnt_rev, self_a, b_a1, rby2_Wl, rev2_Wr)
  r2_tbls, self_a2 = k2a[:nch2], k2a[-1]

  s_rby2, s_rev2 = _sc_agg(
      ((r2_tbls, rby_src, rby_dst, _RPT_U),
       (r2v_tbls, rv_src, rv_dst, _RPT_A)),
      zeros_u, s_ff2, "sc_l2_rest")

  u2 = _k3_user(s_ff2, s_rby2, cnt_ff, cnt_rby, self_u2, b_u2)
  a2 = _k3_app(s_rev2, cnt_rev, self_a2, b_a2)
  return u2, a2


# speedup vs baseline: 2.9785x; 1.0162x over previous
import functools

import jax
import jax.numpy as jnp
from jax import lax
from jax.experimental import pallas as pl
from jax.experimental.pallas import tpu as pltpu
from jax.experimental.pallas import tpu_sc as plsc

_NU, _NA = 50000, 10000
_H, _O = 128, 64
_NU_P, _NA_P = 50048, 10016
_NC, _NS = 2, 16
_NT = _NC * _NS
_K = 128
_W = 16
_RPT_U = _NU_P // _NS
_RPT_A = _NA_P // _NS
_BM = 2000


def _relu(x):
  return jnp.maximum(x, 0.0)


def _full_spec(shape):
  nd = len(shape)
  return pl.BlockSpec(shape, lambda i: (0,) * nd)


def _row_spec(bm, n):
  return pl.BlockSpec((bm, n), lambda i: (i, 0))


def _part_spec(bm, n):
  return pl.BlockSpec((2, bm, n), lambda i: (0, i, 0))


def _chunks(x, n):
  return [x[:, _W * c:_W * (c + 1)] for c in range(n // _W)]


def _k1_user(x_user, Wu, bu, ff1_Wl, rev1_Wl, Wr_u1):
  def body(x_ref, wu_ref, bu_ref, wff_ref, wrev_ref, wru_ref, *outs):
    h = _relu(jnp.dot(x_ref[...], wu_ref[...],
                      preferred_element_type=jnp.float32) + bu_ref[...])
    mff = jnp.dot(h, wff_ref[...], preferred_element_type=jnp.float32)
    mrev = jnp.dot(h, wrev_ref[...], preferred_element_type=jnp.float32)
    for o, val in zip(outs, _chunks(mff, _H) + _chunks(mrev, _H)
                      + [jnp.dot(h, wru_ref[...],
                                 preferred_element_type=jnp.float32)]):
      o[...] = val

  nb = _NU // _BM
  nch = _H // _W
  return pl.pallas_call(
      body,
      grid=(nb,),
      in_specs=[
          _row_spec(_BM, _H), _full_spec((_H, _H)), _full_spec((1, _H)),
          _full_spec((_H, _H)), _full_spec((_H, _H)), _full_spec((_H, _H)),
      ],
      out_specs=[_row_spec(_BM, _W)] * (2 * nch) + [_row_spec(_BM, _H)],
      out_shape=[jax.ShapeDtypeStruct((_NU, _W), jnp.float32)] * (2 * nch)
      + [jax.ShapeDtypeStruct((_NU, _H), jnp.float32)],
  )(x_user, Wu, bu, ff1_Wl, rev1_Wl, Wr_u1)


def _k1_app(x_app, Wa, ba, rby1_Wl, rev1_Wr):
  def body(x_ref, wa_ref, ba_ref, wrby_ref, wsa_ref, *outs):
    h = _relu(jnp.dot(x_ref[...], wa_ref[...],
                      preferred_element_type=jnp.float32) + ba_ref[...])
    m = jnp.dot(h, wrby_ref[...], preferred_element_type=jnp.float32)
    for o, val in zip(outs, _chunks(m, _H)
                      + [jnp.dot(h, wsa_ref[...],
                                 preferred_element_type=jnp.float32)]):
      o[...] = val

  nb = _NA // _BM
  nch = _H // _W
  return pl.pallas_call(
      body,
      grid=(nb,),
      in_specs=[
          _row_spec(_BM, 256), _full_spec((256, _H)), _full_spec((1, _H)),
          _full_spec((_H, _H)), _full_spec((_H, _H)),
      ],
      out_specs=[_row_spec(_BM, _W)] * nch + [_row_spec(_BM, _H)],
      out_shape=[jax.ShapeDtypeStruct((_NA, _W), jnp.float32)] * nch
      + [jax.ShapeDtypeStruct((_NA, _H), jnp.float32)],
  )(x_app, Wa, ba, rby1_Wl, rev1_Wr)


def _inv_from_parts(c_ref):
  cnt = c_ref[0, :, 0:1] + c_ref[1, :, 0:1]
  return 1.0 / jnp.maximum(cnt, 1.0)


def _k2_user(s_ff, s_rby, cnt_ff, cnt_rby, self_u, b_u1, ff2_Wl, rev2_Wl, Wr_u2):
  def body(sff_ref, srby_ref, cff_ref, crby_ref, self_ref, b_ref,
           wff_ref, wrev_ref, wru_ref, *outs):
    sff = sff_ref[0] + sff_ref[1]
    srby = srby_ref[0] + srby_ref[1]
    u = _relu(sff * _inv_from_parts(cff_ref) + srby * _inv_from_parts(crby_ref)
              + b_ref[...] + self_ref[...])
    mff = jnp.dot(u, wff_ref[...], preferred_element_type=jnp.float32)
    mrev = jnp.dot(u, wrev_ref[...], preferred_element_type=jnp.float32)
    for o, val in zip(outs, _chunks(mff, _O) + _chunks(mrev, _O)
                      + [jnp.dot(u, wru_ref[...],
                                 preferred_element_type=jnp.float32)]):
      o[...] = val

  nb = _NU // _BM
  nch = _O // _W
  return pl.pallas_call(
      body,
      grid=(nb,),
      in_specs=[
          _part_spec(_BM, _H), _part_spec(_BM, _H),
          _part_spec(_BM, 16), _part_spec(_BM, 16),
          _row_spec(_BM, _H), _full_spec((1, _H)),
          _full_spec((_H, _O)), _full_spec((_H, _O)), _full_spec((_H, _O)),
      ],
      out_specs=[_row_spec(_BM, _W)] * (2 * nch) + [_row_spec(_BM, _O)],
      out_shape=[jax.ShapeDtypeStruct((_NU, _W), jnp.float32)] * (2 * nch)
      + [jax.ShapeDtypeStruct((_NU, _O), jnp.float32)],
  )(s_ff, s_rby, cnt_ff, cnt_rby, self_u, b_u1, ff2_Wl, rev2_Wl, Wr_u2)


def _k2_app(s_rev, cnt_rev, self_a, b_a1, rby2_Wl, rev2_Wr):
  def body(s_ref, c_ref, self_ref, b_ref, wrby_ref, wsa_ref, *outs):
    s = s_ref[0] + s_ref[1]
    a = _relu(s * _inv_from_parts(c_ref) + b_ref[...] + self_ref[...])
    m = jnp.dot(a, wrby_ref[...], preferred_element_type=jnp.float32)
    for o, val in zip(outs, _chunks(m, _O)
                      + [jnp.dot(a, wsa_ref[...],
                                 preferred_element_type=jnp.float32)]):
      o[...] = val

  nb = _NA // _BM
  nch = _O // _W
  return pl.pallas_call(
      body,
      grid=(nb,),
      in_specs=[
          _part_spec(_BM, _H), _part_spec(_BM, 16),
          _row_spec(_BM, _H), _full_spec((1, _H)),
          _full_spec((_H, _O)), _full_spec((_H, _O)),
      ],
      out_specs=[_row_spec(_BM, _W)] * nch + [_row_spec(_BM, _O)],
      out_shape=[jax.ShapeDtypeStruct((_NA, _W), jnp.float32)] * nch
      + [jax.ShapeDtypeStruct((_NA, _O), jnp.float32)],
  )(s_rev, cnt_rev, self_a, b_a1, rby2_Wl, rev2_Wr)


def _k3_user(s_ff2, s_rby2, cnt_ff, cnt_rby, self_u2, b_u2):
  def body(sff_ref, srby_ref, cff_ref, crby_ref, self_ref, b_ref, out_ref):
    sff = sff_ref[0] + sff_ref[1]
    srby = srby_ref[0] + srby_ref[1]
    out_ref[...] = _relu(
        sff * _inv_from_parts(cff_ref) + srby * _inv_from_parts(crby_ref)
        + b_ref[...] + self_ref[...])

  nb = _NU // _BM
  return pl.pallas_call(
      body,
      grid=(nb,),
      in_specs=[
          _part_spec(_BM, _O), _part_spec(_BM, _O),
          _part_spec(_BM, 16), _part_spec(_BM, 16),
          _row_spec(_BM, _O), _full_spec((1, _O)),
      ],
      out_specs=_row_spec(_BM, _O),
      out_shape=jax.ShapeDtypeStruct((_NU, _O), jnp.float32),
  )(s_ff2, s_rby2, cnt_ff, cnt_rby, self_u2, b_u2)


def _k3_app(s_rev2, cnt_rev, self_a2, b_a2):
  def body(s_ref, c_ref, self_ref, b_ref, out_ref):
    s = s_ref[0] + s_ref[1]
    out_ref[...] = _relu(
        s * _inv_from_parts(c_ref) + b_ref[...] + self_ref[...])

  nb = _NA // _BM
  return pl.pallas_call(
      body,
      grid=(nb,),
      in_specs=[
          _part_spec(_BM, _O), _part_spec(_BM, 16),
          _row_spec(_BM, _O), _full_spec((1, _O)),
      ],
      out_specs=_row_spec(_BM, _O),
      out_shape=jax.ShapeDtypeStruct((_NA, _O), jnp.float32),
  )(s_rev2, cnt_rev, self_a2, b_a2)


_MESH = plsc.VectorSubcoreMesh(core_axis_name="c", subcore_axis_name="s")
_SC_PARAMS = pltpu.CompilerParams(use_tc_tiling_on_sc=False)


def _tile_ids():
  cid = lax.axis_index("c")
  sid = lax.axis_index("s")
  return cid, sid, cid * _NS + sid


def _edge_pipe(tbl_h, sidx2, didx2, rows, acc, gsems, ssems, ipt):
  pltpu.async_copy(tbl_h.at[sidx2.at[0]], rows[0], gsems[0])
  pltpu.async_copy(tbl_h.at[sidx2.at[1]], rows[1], gsems[1])
  nquads = ipt // 4

  def quad(q, carry):
    for k in range(4):
      j = q * 4 + k
      ge, se = gsems[k % 2], ssems[k % 2]
      buf, nbuf = rows[k], rows[(k + 2) % 4]

      @pl.when(j >= 2)
      def _wait_scatter():
        pltpu.make_async_copy(nbuf, acc.at[didx2.at[0]], se).wait()

      pltpu.make_async_copy(tbl_h.at[sidx2.at[0]], buf, ge).wait()

      @pl.when(j + 2 < ipt)
      def _next_gather():
        pltpu.async_copy(tbl_h.at[sidx2.at[j + 2]], nbuf, ge)

      pltpu.async_copy(buf, acc.at[didx2.at[j]], se, add=True)
    return carry

  lax.fori_loop(0, nquads, quad, 0)
  pltpu.make_async_copy(rows[0], acc.at[didx2.at[0]], ssems[0]).wait()
  pltpu.make_async_copy(rows[1], acc.at[didx2.at[0]], ssems[1]).wait()


def _count_pipe(didx2, ones_v, acc, ssem, ipt):
  def body(j, carry):
    @pl.when(j >= 2)
    def _wait():
      pltpu.make_async_copy(ones_v, acc.at[didx2.at[0]], ssem).wait()

    pltpu.async_copy(ones_v, acc.at[didx2.at[j]], ssem, add=True)
    return carry

  lax.fori_loop(0, ipt, body, 0)
  pltpu.make_async_copy(ones_v, acc.at[didx2.at[0]], ssem).wait()
  pltpu.make_async_copy(ones_v, acc.at[didx2.at[0]], ssem).wait()


def _sc_counts(ff_dst, rby_dst, rev_dst, zeros_h, ones_h):
  ipt_ff = ff_dst.shape[0] // _NT
  ipt_rv = rev_dst.shape[0] // _NT

  @functools.partial(
      pl.kernel,
      out_type=[
          jax.ShapeDtypeStruct((_NC, _NU_P, 16), jnp.float32),
          jax.ShapeDtypeStruct((_NC, _NU_P, 16), jnp.float32),
          jax.ShapeDtypeStruct((_NC, _NA_P, 16), jnp.float32),
      ],
      mesh=_MESH,
      scratch_types=[
          pltpu.VMEM((max(ipt_ff, ipt_rv), _K), jnp.int32),
          pltpu.VMEM((_K, 16), jnp.float32),
          pltpu.VMEM((_RPT_U, 16), jnp.float32),
          pltpu.VMEM_SHARED((_NU_P, 16), jnp.float32),
          pltpu.SemaphoreType.DMA,
      ],
      compiler_params=_SC_PARAMS,
      name="sc_counts",
  )
  def k(ffd_h, rbyd_h, revd_h, z_h, o_h, cf_h, cr_h, ca_h,
        didx2, ones_v, zero_v, acc, ssem):
    cid, sid, wid = _tile_ids()
    pltpu.sync_copy(z_h, zero_v)
    pltpu.sync_copy(o_h, ones_v)
    for dst_h, out_h, rpt, ipt in (
        (ffd_h, cf_h, _RPT_U, ipt_ff), (rbyd_h, cr_h, _RPT_U, ipt_rv),
        (revd_h, ca_h, _RPT_A, ipt_rv)):
      pltpu.sync_copy(dst_h.at[pl.ds(wid * ipt, ipt)], didx2.at[pl.ds(0, ipt)])
      pltpu.sync_copy(zero_v.at[pl.ds(0, rpt)], acc.at[pl.ds(sid * rpt, rpt)])
      plsc.subcore_barrier()
      _count_pipe(didx2, ones_v, acc, ssem, ipt)
      plsc.subcore_barrier()
      pltpu.sync_copy(acc.at[pl.ds(sid * rpt, rpt)],
                      out_h.at[cid, pl.ds(sid * rpt, rpt)])
      plsc.subcore_barrier()

  return k(ff_dst, rby_dst, rev_dst, zeros_h, ones_h)


def _sc_agg(groups, zeros_h, token, name):
  counts = [len(tbls) for tbls, _, _, _ in groups]
  ipts = [src2.shape[0] // _NT for _, src2, _, _ in groups]
  ipt_max = max(ipts)
  ntbl = sum(counts)
  ng = len(groups)

  @functools.partial(
      pl.kernel,
      out_type=[
          jax.ShapeDtypeStruct((_NC, rpt * _NS, _W * len(tbls)), jnp.float32)
          for tbls, _, _, rpt in groups
      ],
      mesh=_MESH,
      scratch_types=[
          pltpu.VMEM((ipt_max, _K), jnp.int32),
          pltpu.VMEM((ipt_max, _K), jnp.int32),
          pltpu.VMEM((_K, _W), jnp.float32),
          pltpu.VMEM((_K, _W), jnp.float32),
          pltpu.VMEM((_K, _W), jnp.float32),
          pltpu.VMEM((_K, _W), jnp.float32),
          pltpu.VMEM((_RPT_U, _W), jnp.float32),
          pltpu.VMEM_SHARED((_NU_P, _W), jnp.float32),
          pltpu.SemaphoreType.DMA,
          pltpu.SemaphoreType.DMA,
          pltpu.SemaphoreType.DMA,
          pltpu.SemaphoreType.DMA,
      ],
      compiler_params=_SC_PARAMS,
      name=name,
  )
  def k(*refs):
    pos = 0
    tbl_refs = []
    for n in counts:
      tbl_refs.append(refs[pos:pos + n])
      pos += n
    edge_refs = refs[pos:pos + 2 * ng]
    pos += 2 * ng
    z_h = refs[pos]
    out_refs = refs[pos + 2:pos + 2 + ng]
    (sidx2, didx2, rows0, rows1, rows2, rows3, zero_v, acc,
     gsem0, gsem1, ssem0, ssem1) = refs[pos + 2 + ng:]
    cid, sid, wid = _tile_ids()
    pltpu.sync_copy(z_h, zero_v)

    def subpass(tbl, out_h, col, rpt, ipt):
      pltpu.sync_copy(zero_v.at[pl.ds(0, rpt)], acc.at[pl.ds(sid * rpt, rpt)])
      plsc.subcore_barrier()
      _edge_pipe(tbl, sidx2, didx2, (rows0, rows1, rows2, rows3), acc,
                 (gsem0, gsem1), (ssem0, ssem1), ipt)
      plsc.subcore_barrier()
      pltpu.sync_copy(acc.at[pl.ds(sid * rpt, rpt)],
                      out_h.at[cid, pl.ds(sid * rpt, rpt), pl.ds(col, _W)])
      plsc.subcore_barrier()

    for g in range(ng):
      src_h, dst_h = edge_refs[2 * g], edge_refs[2 * g + 1]
      rpt, ipt = groups[g][3], ipts[g]
      pltpu.sync_copy(src_h.at[pl.ds(wid * ipt, ipt)], sidx2.at[pl.ds(0, ipt)])
      pltpu.sync_copy(dst_h.at[pl.ds(wid * ipt, ipt)], didx2.at[pl.ds(0, ipt)])
      for c, tbl in enumerate(tbl_refs[g]):
        subpass(tbl, out_refs[g], _W * c, rpt, ipt)

  args = []
  for tbls, _, _, _ in groups:
    args.extend(tbls)
  for _, src2, dst2, _ in groups:
    args.extend((src2, dst2))
  args.extend((zeros_h, token))
  out = k(*args)
  return tuple(out) if isinstance(out, (list, tuple)) else (out,)


def _pad_edges(src, dst, junk, njunk):
  e = src.shape[0]
  ep = -(-e // (4 * _NT * _K)) * (4 * _NT * _K)
  pad = jnp.arange(ep - e, dtype=jnp.int32)
  src_p = jnp.concatenate([src.astype(jnp.int32), pad % 997])
  dst_p = jnp.concatenate([dst.astype(jnp.int32), junk + pad % njunk])
  return src_p.reshape(-1, _K), dst_p.reshape(-1, _K)


def kernel(x_user, x_app, edge_ff, rev_src, rev_dst, Wu, bu, Wa, ba,
           ff1_Wl, ff1_bl, ff1_Wr, rev1_Wl, rev1_bl, rev1_Wr,
           rby1_Wl, rby1_bl, rby1_Wr, ff2_Wl, ff2_bl, ff2_Wr,
           rev2_Wl, rev2_bl, rev2_Wr, rby2_Wl, rby2_bl, rby2_Wr):
  ff_src, ff_dst = _pad_edges(edge_ff[0], edge_ff[1], _NU, _NU_P - _NU)
  rby_src, rby_dst = _pad_edges(rev_dst, rev_src, _NU, _NU_P - _NU)
  rv_src, rv_dst = _pad_edges(rev_src, rev_dst, _NA, _NA_P - _NA)

  zeros_u = jnp.zeros((_RPT_U, _W), jnp.float32)
  ones_h = jnp.ones((_K, 16), jnp.float32)

  Wr_u1 = ff1_Wr + rby1_Wr
  Wr_u2 = ff2_Wr + rby2_Wr
  b_u1 = (ff1_bl + rby1_bl).reshape(1, _H)
  b_u2 = (ff2_bl + rby2_bl).reshape(1, _O)
  b_a1 = rev1_bl.reshape(1, _H)
  b_a2 = rev2_bl.reshape(1, _O)

  cnt_ff, cnt_rby, cnt_rev = _sc_counts(ff_dst, rby_dst, rv_dst,
                                        zeros_u, ones_h)

  nch = _H // _W
  k1u = _k1_user(x_user, Wu, bu.reshape(1, _H), ff1_Wl, rev1_Wl, Wr_u1)
  mff_tbls, mrev_tbls, self_u = k1u[:nch], k1u[nch:2 * nch], k1u[-1]
  k1a = _k1_app(x_app, Wa, ba.reshape(1, _H), rby1_Wl, rev1_Wr)
  mrby_tbls, self_a = k1a[:nch], k1a[-1]

  s_ff, s_rby = _sc_agg(
      ((mff_tbls, ff_src, ff_dst, _RPT_U),
       (mrby_tbls, rby_src, rby_dst, _RPT_U)),
      zeros_u, cnt_rev, "sc_l1_users")
  (s_rev,) = _sc_agg(((mrev_tbls, rv_src, rv_dst, _RPT_A),),
                     zeros_u, s_rby, "sc_l1_apps")

  nch2 = _O // _W
  k2u = _k2_user(s_ff, s_rby, cnt_ff, cnt_rby, self_u, b_u1,
                 ff2_Wl, rev2_Wl, Wr_u2)
  f2_tbls, r2v_tbls, self_u2 = k2u[:nch2], k2u[nch2:2 * nch2], k2u[-1]
  (s_ff2,) = _sc_agg(((f2_tbls, ff_src, ff_dst, _RPT_U),),
                     zeros_u, s_rev, "sc_l2_ff")
  k2a = _k2_app(s_rev, cnt_rev, self_a, b_a1, rby2_Wl, rev2_Wr)
  r2_tbls, self_a2 = k2a[:nch2], k2a[-1]

  (s_rby2,) = _sc_agg(((r2_tbls, rby_src, rby_dst, _RPT_U),),
                      zeros_u, s_ff2, "sc_l2_rby")
  (s_rev2,) = _sc_agg(((r2v_tbls, rv_src, rv_dst, _RPT_A),),
                      zeros_u, s_rby2, "sc_l2_rev")

  u2 = _k3_user(s_ff2, s_rby2, cnt_ff, cnt_rby, self_u2, b_u2)
  a2 = _k3_app(s_rev2, cnt_rev, self_a2, b_a2)
  return u2, a2


# speedup vs baseline: 2.9825x; 1.0013x over previous
import functools

import jax
import jax.numpy as jnp
from jax import lax
from jax.experimental import pallas as pl
from jax.experimental.pallas import tpu as pltpu
from jax.experimental.pallas import tpu_sc as plsc

_NU, _NA = 50000, 10000
_H, _O = 128, 64
_NU_P, _NA_P = 50048, 10016
_NC, _NS = 2, 16
_NT = _NC * _NS
_K = 128
_W = 16
_RPT_U = _NU_P // _NS
_RPT_A = _NA_P // _NS
_BM = 2000


def _relu(x):
  return jnp.maximum(x, 0.0)


def _full_spec(shape):
  nd = len(shape)
  return pl.BlockSpec(shape, lambda i: (0,) * nd)


def _row_spec(bm, n):
  return pl.BlockSpec((bm, n), lambda i: (i, 0))


def _part_spec(bm, n):
  return pl.BlockSpec((2, bm, n), lambda i: (0, i, 0))


def _chunks(x, n):
  return [x[:, _W * c:_W * (c + 1)] for c in range(n // _W)]


def _k1_user(x_user, Wu, bu, ff1_Wl, rev1_Wl, Wr_u1):
  def body(x_ref, wu_ref, bu_ref, wff_ref, wrev_ref, wru_ref, *outs):
    h = _relu(jnp.dot(x_ref[...], wu_ref[...],
                      preferred_element_type=jnp.float32) + bu_ref[...])
    mff = jnp.dot(h, wff_ref[...], preferred_element_type=jnp.float32)
    mrev = jnp.dot(h, wrev_ref[...], preferred_element_type=jnp.float32)
    for o, val in zip(outs, _chunks(mff, _H) + _chunks(mrev, _H)
                      + [jnp.dot(h, wru_ref[...],
                                 preferred_element_type=jnp.float32)]):
      o[...] = val

  nb = _NU // _BM
  nch = _H // _W
  return pl.pallas_call(
      body,
      grid=(nb,),
      in_specs=[
          _row_spec(_BM, _H), _full_spec((_H, _H)), _full_spec((1, _H)),
          _full_spec((_H, _H)), _full_spec((_H, _H)), _full_spec((_H, _H)),
      ],
      out_specs=[_row_spec(_BM, _W)] * (2 * nch) + [_row_spec(_BM, _H)],
      out_shape=[jax.ShapeDtypeStruct((_NU, _W), jnp.float32)] * (2 * nch)
      + [jax.ShapeDtypeStruct((_NU, _H), jnp.float32)],
  )(x_user, Wu, bu, ff1_Wl, rev1_Wl, Wr_u1)


def _k1_app(x_app, Wa, ba, rby1_Wl, rev1_Wr):
  def body(x_ref, wa_ref, ba_ref, wrby_ref, wsa_ref, *outs):
    h = _relu(jnp.dot(x_ref[...], wa_ref[...],
                      preferred_element_type=jnp.float32) + ba_ref[...])
    m = jnp.dot(h, wrby_ref[...], preferred_element_type=jnp.float32)
    for o, val in zip(outs, _chunks(m, _H)
                      + [jnp.dot(h, wsa_ref[...],
                                 preferred_element_type=jnp.float32)]):
      o[...] = val

  nb = _NA // _BM
  nch = _H // _W
  return pl.pallas_call(
      body,
      grid=(nb,),
      in_specs=[
          _row_spec(_BM, 256), _full_spec((256, _H)), _full_spec((1, _H)),
          _full_spec((_H, _H)), _full_spec((_H, _H)),
      ],
      out_specs=[_row_spec(_BM, _W)] * nch + [_row_spec(_BM, _H)],
      out_shape=[jax.ShapeDtypeStruct((_NA, _W), jnp.float32)] * nch
      + [jax.ShapeDtypeStruct((_NA, _H), jnp.float32)],
  )(x_app, Wa, ba, rby1_Wl, rev1_Wr)


def _inv_from_parts(c_ref):
  cnt = c_ref[0, :, 0:1] + c_ref[1, :, 0:1]
  return 1.0 / jnp.maximum(cnt, 1.0)


def _k2_user(s_ff, s_rby, cnt_ff, cnt_rby, self_u, b_u1, ff2_Wl, rev2_Wl, Wr_u2):
  def body(sff_ref, srby_ref, cff_ref, crby_ref, self_ref, b_ref,
           wff_ref, wrev_ref, wru_ref, *outs):
    sff = sff_ref[0] + sff_ref[1]
    srby = srby_ref[0] + srby_ref[1]
    u = _relu(sff * _inv_from_parts(cff_ref) + srby * _inv_from_parts(crby_ref)
              + b_ref[...] + self_ref[...])
    mff = jnp.dot(u, wff_ref[...], preferred_element_type=jnp.float32)
    mrev = jnp.dot(u, wrev_ref[...], preferred_element_type=jnp.float32)
    for o, val in zip(outs, _chunks(mff, _O) + _chunks(mrev, _O)
                      + [jnp.dot(u, wru_ref[...],
                                 preferred_element_type=jnp.float32)]):
      o[...] = val

  nb = _NU // _BM
  nch = _O // _W
  return pl.pallas_call(
      body,
      grid=(nb,),
      in_specs=[
          _part_spec(_BM, _H), _part_spec(_BM, _H),
          _part_spec(_BM, 16), _part_spec(_BM, 16),
          _row_spec(_BM, _H), _full_spec((1, _H)),
          _full_spec((_H, _O)), _full_spec((_H, _O)), _full_spec((_H, _O)),
      ],
      out_specs=[_row_spec(_BM, _W)] * (2 * nch) + [_row_spec(_BM, _O)],
      out_shape=[jax.ShapeDtypeStruct((_NU, _W), jnp.float32)] * (2 * nch)
      + [jax.ShapeDtypeStruct((_NU, _O), jnp.float32)],
  )(s_ff, s_rby, cnt_ff, cnt_rby, self_u, b_u1, ff2_Wl, rev2_Wl, Wr_u2)


def _k2_app(s_rev, cnt_rev, self_a, b_a1, rby2_Wl, rev2_Wr):
  def body(s_ref, c_ref, self_ref, b_ref, wrby_ref, wsa_ref, *outs):
    s = s_ref[0] + s_ref[1]
    a = _relu(s * _inv_from_parts(c_ref) + b_ref[...] + self_ref[...])
    m = jnp.dot(a, wrby_ref[...], preferred_element_type=jnp.float32)
    for o, val in zip(outs, _chunks(m, _O)
                      + [jnp.dot(a, wsa_ref[...],
                                 preferred_element_type=jnp.float32)]):
      o[...] = val

  nb = _NA // _BM
  nch = _O // _W
  return pl.pallas_call(
      body,
      grid=(nb,),
      in_specs=[
          _part_spec(_BM, _H), _part_spec(_BM, 16),
          _row_spec(_BM, _H), _full_spec((1, _H)),
          _full_spec((_H, _O)), _full_spec((_H, _O)),
      ],
      out_specs=[_row_spec(_BM, _W)] * nch + [_row_spec(_BM, _O)],
      out_shape=[jax.ShapeDtypeStruct((_NA, _W), jnp.float32)] * nch
      + [jax.ShapeDtypeStruct((_NA, _O), jnp.float32)],
  )(s_rev, cnt_rev, self_a, b_a1, rby2_Wl, rev2_Wr)


def _k3_user(s_ff2, s_rby2, cnt_ff, cnt_rby, self_u2, b_u2):
  def body(sff_ref, srby_ref, cff_ref, crby_ref, self_ref, b_ref, out_ref):
    sff = sff_ref[0] + sff_ref[1]
    srby = srby_ref[0] + srby_ref[1]
    out_ref[...] = _relu(
        sff * _inv_from_parts(cff_ref) + srby * _inv_from_parts(crby_ref)
        + b_ref[...] + self_ref[...])

  nb = _NU // _BM
  return pl.pallas_call(
      body,
      grid=(nb,),
      in_specs=[
          _part_spec(_BM, _O), _part_spec(_BM, _O),
          _part_spec(_BM, 16), _part_spec(_BM, 16),
          _row_spec(_BM, _O), _full_spec((1, _O)),
      ],
      out_specs=_row_spec(_BM, _O),
      out_shape=jax.ShapeDtypeStruct((_NU, _O), jnp.float32),
  )(s_ff2, s_rby2, cnt_ff, cnt_rby, self_u2, b_u2)


def _k3_app(s_rev2, cnt_rev, self_a2, b_a2):
  def body(s_ref, c_ref, self_ref, b_ref, out_ref):
    s = s_ref[0] + s_ref[1]
    out_ref[...] = _relu(
        s * _inv_from_parts(c_ref) + b_ref[...] + self_ref[...])

  nb = _NA // _BM
  return pl.pallas_call(
      body,
      grid=(nb,),
      in_specs=[
          _part_spec(_BM, _O), _part_spec(_BM, 16),
          _row_spec(_BM, _O), _full_spec((1, _O)),
      ],
      out_specs=_row_spec(_BM, _O),
      out_shape=jax.ShapeDtypeStruct((_NA, _O), jnp.float32),
  )(s_rev2, cnt_rev, self_a2, b_a2)


_MESH = plsc.VectorSubcoreMesh(core_axis_name="c", subcore_axis_name="s",
                               num_cores=_NC, num_subcores=_NS)
_SC_PARAMS = pltpu.CompilerParams(use_tc_tiling_on_sc=False)


def _tile_ids():
  cid = lax.axis_index("c")
  sid = lax.axis_index("s")
  return cid, sid, cid * _NS + sid


def _edge_pipe(tbl_h, sidx2, didx2, rows, acc, gsems, ssems, ipt):
  pltpu.async_copy(tbl_h.at[sidx2.at[0]], rows[0], gsems[0])
  pltpu.async_copy(tbl_h.at[sidx2.at[1]], rows[1], gsems[1])
  nquads = ipt // 4

  def quad(q, carry):
    for k in range(4):
      j = q * 4 + k
      ge, se = gsems[k % 2], ssems[k % 2]
      buf, nbuf = rows[k], rows[(k + 2) % 4]

      @pl.when(j >= 2)
      def _wait_scatter():
        pltpu.make_async_copy(nbuf, acc.at[didx2.at[0]], se).wait()

      pltpu.make_async_copy(tbl_h.at[sidx2.at[0]], buf, ge).wait()

      @pl.when(j + 2 < ipt)
      def _next_gather():
        pltpu.async_copy(tbl_h.at[sidx2.at[j + 2]], nbuf, ge)

      pltpu.async_copy(buf, acc.at[didx2.at[j]], se, add=True)
    return carry

  lax.fori_loop(0, nquads, quad, 0)
  pltpu.make_async_copy(rows[0], acc.at[didx2.at[0]], ssems[0]).wait()
  pltpu.make_async_copy(rows[1], acc.at[didx2.at[0]], ssems[1]).wait()


def _count_pipe(didx2, ones_v, acc, ssem, ipt):
  def body(j, carry):
    @pl.when(j >= 2)
    def _wait():
      pltpu.make_async_copy(ones_v, acc.at[didx2.at[0]], ssem).wait()

    pltpu.async_copy(ones_v, acc.at[didx2.at[j]], ssem, add=True)
    return carry

  lax.fori_loop(0, ipt, body, 0)
  pltpu.make_async_copy(ones_v, acc.at[didx2.at[0]], ssem).wait()
  pltpu.make_async_copy(ones_v, acc.at[didx2.at[0]], ssem).wait()


def _sc_counts(ff_dst, rby_dst, rev_dst, zeros_h, ones_h):
  ipt_ff = ff_dst.shape[0] // _NT
  ipt_rv = rev_dst.shape[0] // _NT

  @functools.partial(
      pl.kernel,
      out_type=[
          jax.ShapeDtypeStruct((_NC, _NU_P, 16), jnp.float32),
          jax.ShapeDtypeStruct((_NC, _NU_P, 16), jnp.float32),
          jax.ShapeDtypeStruct((_NC, _NA_P, 16), jnp.float32),
      ],
      mesh=_MESH,
      scratch_types=[
          pltpu.VMEM((max(ipt_ff, ipt_rv), _K), jnp.int32),
          pltpu.VMEM((_K, 16), jnp.float32),
          pltpu.VMEM((_RPT_U, 16), jnp.float32),
          pltpu.VMEM_SHARED((_NU_P, 16), jnp.float32),
          pltpu.SemaphoreType.DMA,
      ],
      compiler_params=_SC_PARAMS,
      name="sc_counts",
  )
  def k(ffd_h, rbyd_h, revd_h, z_h, o_h, cf_h, cr_h, ca_h,
        didx2, ones_v, zero_v, acc, ssem):
    cid, sid, wid = _tile_ids()
    pltpu.sync_copy(z_h, zero_v)
    pltpu.sync_copy(o_h, ones_v)
    for dst_h, out_h, rpt, ipt in (
        (ffd_h, cf_h, _RPT_U, ipt_ff), (rbyd_h, cr_h, _RPT_U, ipt_rv),
        (revd_h, ca_h, _RPT_A, ipt_rv)):
      pltpu.sync_copy(dst_h.at[pl.ds(wid * ipt, ipt)], didx2.at[pl.ds(0, ipt)])
      pltpu.sync_copy(zero_v.at[pl.ds(0, rpt)], acc.at[pl.ds(sid * rpt, rpt)])
      plsc.subcore_barrier()
      _count_pipe(didx2, ones_v, acc, ssem, ipt)
      plsc.subcore_barrier()
      pltpu.sync_copy(acc.at[pl.ds(sid * rpt, rpt)],
                      out_h.at[cid, pl.ds(sid * rpt, rpt)])
      plsc.subcore_barrier()

  return k(ff_dst, rby_dst, rev_dst, zeros_h, ones_h)


def _sc_agg(groups, zeros_h, token, name):
  counts = [len(tbls) for tbls, _, _, _ in groups]
  ipts = [src2.shape[0] // _NT for _, src2, _, _ in groups]
  ipt_max = max(ipts)
  ntbl = sum(counts)
  ng = len(groups)

  @functools.partial(
      pl.kernel,
      out_type=[
          jax.ShapeDtypeStruct((_NC, rpt * _NS, _W * len(tbls)), jnp.float32)
          for tbls, _, _, rpt in groups
      ],
      mesh=_MESH,
      scratch_types=[
          pltpu.VMEM((ipt_max, _K), jnp.int32),
          pltpu.VMEM((ipt_max, _K), jnp.int32),
          pltpu.VMEM((_K, _W), jnp.float32),
          pltpu.VMEM((_K, _W), jnp.float32),
          pltpu.VMEM((_K, _W), jnp.float32),
          pltpu.VMEM((_K, _W), jnp.float32),
          pltpu.VMEM((_RPT_U, _W), jnp.float32),
          pltpu.VMEM_SHARED((_NU_P, _W), jnp.float32),
          pltpu.SemaphoreType.DMA,
          pltpu.SemaphoreType.DMA,
          pltpu.SemaphoreType.DMA,
          pltpu.SemaphoreType.DMA,
      ],
      compiler_params=_SC_PARAMS,
      name=name,
  )
  def k(*refs):
    pos = 0
    tbl_refs = []
    for n in counts:
      tbl_refs.append(refs[pos:pos + n])
      pos += n
    edge_refs = refs[pos:pos + 2 * ng]
    pos += 2 * ng
    z_h = refs[pos]
    out_refs = refs[pos + 2:pos + 2 + ng]
    (sidx2, didx2, rows0, rows1, rows2, rows3, zero_v, acc,
     gsem0, gsem1, ssem0, ssem1) = refs[pos + 2 + ng:]
    cid, sid, wid = _tile_ids()
    pltpu.sync_copy(z_h, zero_v)

    def subpass(tbl, out_h, col, rpt, ipt):
      pltpu.sync_copy(zero_v.at[pl.ds(0, rpt)], acc.at[pl.ds(sid * rpt, rpt)])
      plsc.subcore_barrier()
      _edge_pipe(tbl, sidx2, didx2, (rows0, rows1, rows2, rows3), acc,
                 (gsem0, gsem1), (ssem0, ssem1), ipt)
      plsc.subcore_barrier()
      pltpu.sync_copy(acc.at[pl.ds(sid * rpt, rpt)],
                      out_h.at[cid, pl.ds(sid * rpt, rpt), pl.ds(col, _W)])
      plsc.subcore_barrier()

    for g in range(ng):
      src_h, dst_h = edge_refs[2 * g], edge_refs[2 * g + 1]
      rpt, ipt = groups[g][3], ipts[g]
      pltpu.sync_copy(src_h.at[pl.ds(wid * ipt, ipt)], sidx2.at[pl.ds(0, ipt)])
      pltpu.sync_copy(dst_h.at[pl.ds(wid * ipt, ipt)], didx2.at[pl.ds(0, ipt)])
      for c, tbl in enumerate(tbl_refs[g]):
        subpass(tbl, out_refs[g], _W * c, rpt, ipt)

  args = []
  for tbls, _, _, _ in groups:
    args.extend(tbls)
  for _, src2, dst2, _ in groups:
    args.extend((src2, dst2))
  args.extend((zeros_h, token))
  out = k(*args)
  return tuple(out) if isinstance(out, (list, tuple)) else (out,)


def _pad_edges(src, dst, junk, njunk):
  e = src.shape[0]
  ep = -(-e // (4 * _NT * _K)) * (4 * _NT * _K)
  pad = jnp.arange(ep - e, dtype=jnp.int32)
  src_p = jnp.concatenate([src.astype(jnp.int32), pad % 997])
  dst_p = jnp.concatenate([dst.astype(jnp.int32), junk + pad % njunk])
  return src_p.reshape(-1, _K), dst_p.reshape(-1, _K)


def kernel(x_user, x_app, edge_ff, rev_src, rev_dst, Wu, bu, Wa, ba,
           ff1_Wl, ff1_bl, ff1_Wr, rev1_Wl, rev1_bl, rev1_Wr,
           rby1_Wl, rby1_bl, rby1_Wr, ff2_Wl, ff2_bl, ff2_Wr,
           rev2_Wl, rev2_bl, rev2_Wr, rby2_Wl, rby2_bl, rby2_Wr):
  ff_src, ff_dst = _pad_edges(edge_ff[0], edge_ff[1], _NU, _NU_P - _NU)
  rby_src, rby_dst = _pad_edges(rev_dst, rev_src, _NU, _NU_P - _NU)
  rv_src, rv_dst = _pad_edges(rev_src, rev_dst, _NA, _NA_P - _NA)

  zeros_u = jnp.zeros((_RPT_U, _W), jnp.float32)
  ones_h = jnp.ones((_K, 16), jnp.float32)

  Wr_u1 = ff1_Wr + rby1_Wr
  Wr_u2 = ff2_Wr + rby2_Wr
  b_u1 = (ff1_bl + rby1_bl).reshape(1, _H)
  b_u2 = (ff2_bl + rby2_bl).reshape(1, _O)
  b_a1 = rev1_bl.reshape(1, _H)
  b_a2 = rev2_bl.reshape(1, _O)

  cnt_ff, cnt_rby, cnt_rev = _sc_counts(ff_dst, rby_dst, rv_dst,
                                        zeros_u, ones_h)

  nch = _H // _W
  k1u = _k1_user(x_user, Wu, bu.reshape(1, _H), ff1_Wl, rev1_Wl, Wr_u1)
  mff_tbls, mrev_tbls, self_u = k1u[:nch], k1u[nch:2 * nch], k1u[-1]
  k1a = _k1_app(x_app, Wa, ba.reshape(1, _H), rby1_Wl, rev1_Wr)
  mrby_tbls, self_a = k1a[:nch], k1a[-1]

  s_ff, s_rby = _sc_agg(
      ((mff_tbls, ff_src, ff_dst, _RPT_U),
       (mrby_tbls, rby_src, rby_dst, _RPT_U)),
      zeros_u, cnt_rev, "sc_l1_users")
  (s_rev,) = _sc_agg(((mrev_tbls, rv_src, rv_dst, _RPT_A),),
                     zeros_u, s_rby, "sc_l1_apps")

  nch2 = _O // _W
  k2u = _k2_user(s_ff, s_rby, cnt_ff, cnt_rby, self_u, b_u1,
                 ff2_Wl, rev2_Wl, Wr_u2)
  f2_tbls, r2v_tbls, self_u2 = k2u[:nch2], k2u[nch2:2 * nch2], k2u[-1]
  (s_ff2,) = _sc_agg(((f2_tbls, ff_src, ff_dst, _RPT_U),),
                     zeros_u, s_rev, "sc_l2_ff")
  k2a = _k2_app(s_rev, cnt_rev, self_a, b_a1, rby2_Wl, rev2_Wr)
  r2_tbls, self_a2 = k2a[:nch2], k2a[-1]

  (s_rby2,) = _sc_agg(((r2_tbls, rby_src, rby_dst, _RPT_U),),
                      zeros_u, s_ff2, "sc_l2_rby")
  (s_rev2,) = _sc_agg(((r2v_tbls, rv_src, rv_dst, _RPT_A),),
                      zeros_u, s_rby2, "sc_l2_rev")

  u2 = _k3_user(s_ff2, s_rby2, cnt_ff, cnt_rby, self_u2, b_u2)
  a2 = _k3_app(s_rev2, cnt_rev, self_a2, b_a2)
  return u2, a2
